# Initial kernel scaffold; baseline (speedup 1.0000x reference)
import jax
import jax.numpy as jnp
from jax.experimental import pallas as pl


def kernel(x, edge_index, edge_weight, w1a, b1a, g1, be1, w1b, b1b, w2a, b2a, g2, be2, w2b, b2b):
    raise NotImplementedError("write your pallas kernel here")


# speedup vs baseline: 10.2963x; 10.2963x over previous
import functools

import jax
import jax.numpy as jnp
import numpy as np
from jax import lax
from jax.experimental import pallas as pl
from jax.experimental.pallas import tpu as pltpu
from jax.experimental.pallas import tpu_sc as plsc

_N = 10000
_E = 320000
_DIN = 128
_H = 32
_NC = 2
_NS = 16
_NW = _NC * _NS
_C = 128
_EPW = 10240
_CH = _EPW // _C
_NPAD = 10240
_RPT = _NPAD // _NS
_BN = float(1.0 / np.sqrt(1.0 + 1e-5))

_ROWB = 1000
_GRID = _N // _ROWB


def _segsum_body(y_hbm, src_hbm, dst_hbm, ew_hbm, out_hbm,
                 acc_sh, src_v, dst_v, ew_v, rows0, rows1, zbuf, sem0, sem1):
    cid = lax.axis_index("c")
    sid = lax.axis_index("s")
    wid = sid * _NC + cid

    z16 = jnp.zeros((16,), jnp.float32)

    def _zero_row(i, carry):
        zbuf[i, pl.ds(0, 16)] = z16
        zbuf[i, pl.ds(16, 16)] = z16
        return carry

    lax.fori_loop(0, _RPT, _zero_row, 0)
    pltpu.sync_copy(zbuf, acc_sh.at[pl.ds(sid * _RPT, _RPT)])

    pltpu.sync_copy(src_hbm.at[wid], src_v)
    pltpu.sync_copy(dst_hbm.at[wid], dst_v)
    pltpu.sync_copy(ew_hbm.at[wid], ew_v)

    plsc.subcore_barrier()

    bufs = ((rows0, sem0), (rows1, sem1))

    def _start(j, b):
        rows, sem = bufs[b]
        pltpu.async_copy(y_hbm.at[src_v.at[j]], rows, sem)

    def _wait(j, b):
        rows, sem = bufs[b]
        pltpu.make_async_copy(y_hbm.at[src_v.at[j]], rows, sem).wait()

    def _process(j, b):
        rows, _ = bufs[b]

        def _scale(i, carry):
            e0 = i * 16
            wv = ew_v[j, pl.ds(e0, 16)]
            for k in range(16):
                w = wv[k]
                rows[e0 + k, pl.ds(0, 16)] = rows[e0 + k, pl.ds(0, 16)] * w
                rows[e0 + k, pl.ds(16, 16)] = rows[e0 + k, pl.ds(16, 16)] * w
            return carry

        lax.fori_loop(0, _C // 16, _scale, 0)
        pltpu.sync_copy(rows, acc_sh.at[dst_v.at[j]], add=True)

    _start(0, 0)
    _start(1, 1)

    def _outer(j, carry):
        for b in range(2):
            _wait(j + b, b)
            _process(j + b, b)
            _start(j + b + 2, b)
        return carry

    lax.fori_loop(0, (_CH - 2) // 2, lambda i, c: _outer(i * 2, c), 0)
    for b in range(2):
        _wait(_CH - 2 + b, b)
        _process(_CH - 2 + b, b)

    plsc.subcore_barrier()
    pltpu.sync_copy(
        acc_sh.at[pl.ds(sid * _RPT, _RPT)],
        out_hbm.at[cid, pl.ds(sid * _RPT, _RPT)],
    )


@functools.lru_cache(maxsize=1)
def _build_segsum():
    mesh = plsc.VectorSubcoreMesh(
        core_axis_name="c", subcore_axis_name="s",
        num_cores=_NC, num_subcores=_NS,
    )
    return pl.kernel(
        _segsum_body,
        out_type=jax.ShapeDtypeStruct((_NC, _NPAD, _H), jnp.float32),
        mesh=mesh,
        scratch_types=[
            pltpu.VMEM_SHARED((_NPAD, _H), jnp.float32),
            pltpu.VMEM((_CH, _C), jnp.int32),
            pltpu.VMEM((_CH, _C), jnp.int32),
            pltpu.VMEM((_CH, _C), jnp.float32),
            pltpu.VMEM((_C, _H), jnp.float32),
            pltpu.VMEM((_C, _H), jnp.float32),
            pltpu.VMEM((_RPT, _H), jnp.float32),
            pltpu.SemaphoreType.DMA,
            pltpu.SemaphoreType.DMA,
        ],
        compiler_params=pltpu.CompilerParams(use_tc_tiling_on_sc=False),
    )


def _segsum(y, src, dst, ew):
    return _build_segsum()(y, src, dst, ew)


def _tc1_body(x_ref, w_ref, o_ref):
    o_ref[...] = jnp.dot(x_ref[...], w_ref[...],
                         preferred_element_type=jnp.float32)


def _tc1(x, w1a):
    return pl.pallas_call(
        _tc1_body,
        grid=(_GRID,),
        in_specs=[
            pl.BlockSpec((_ROWB, _DIN), lambda i: (i, 0)),
            pl.BlockSpec((_DIN, _H), lambda i: (0, 0)),
        ],
        out_specs=pl.BlockSpec((_ROWB, _H), lambda i: (i, 0)),
        out_shape=jax.ShapeDtypeStruct((_N, _H), jnp.float32),
    )(x, w1a)


def _tc2_body(p_ref, q_ref, y_ref, b1a_ref, g1_ref, be1_ref, w1b_ref,
              b1b_ref, o_ref):
    t = p_ref[0] + q_ref[0] + y_ref[...] + b1a_ref[...]
    t = t * (g1_ref[...] * _BN) + be1_ref[...]
    t = jnp.maximum(t, 0.0)
    t = jnp.dot(t, w1b_ref[...], preferred_element_type=jnp.float32)
    o_ref[...] = jnp.maximum(t + b1b_ref[...], 0.0)


def _tc2(parts, y1, b1a, g1, be1, w1b, b1b):
    vec = pl.BlockSpec((1, _H), lambda i: (0, 0))
    return pl.pallas_call(
        _tc2_body,
        grid=(_GRID,),
        in_specs=[
            pl.BlockSpec((1, _ROWB, _H), lambda i: (0, i, 0)),
            pl.BlockSpec((1, _ROWB, _H), lambda i: (1, i, 0)),
            pl.BlockSpec((_ROWB, _H), lambda i: (i, 0)),
            vec, vec, vec,
            pl.BlockSpec((_H, _H), lambda i: (0, 0)),
            vec,
        ],
        out_specs=pl.BlockSpec((_ROWB, _H), lambda i: (i, 0)),
        out_shape=jax.ShapeDtypeStruct((_N, _H), jnp.float32),
    )(parts, parts, y1, b1a.reshape(1, _H), g1.reshape(1, _H),
      be1.reshape(1, _H), w1b, b1b.reshape(1, _H))


def _tc3_body(p_ref, q_ref, h_ref, w2a_ref, b2a_ref, g2_ref, be2_ref,
              w2b_ref, b2b_ref, o_ref):
    t = p_ref[0] + q_ref[0] + h_ref[...]
    t = jnp.dot(t, w2a_ref[...], preferred_element_type=jnp.float32)
    t = (t + b2a_ref[...]) * (g2_ref[...] * _BN) + be2_ref[...]
    t = jnp.maximum(t, 0.0)
    t = jnp.dot(t, w2b_ref[...], preferred_element_type=jnp.float32)
    o_ref[...] = t + b2b_ref[...]


def _tc3(parts, h1, w2a, b2a, g2, be2, w2b, b2b):
    vec = pl.BlockSpec((1, _DIN), lambda i: (0, 0))
    return pl.pallas_call(
        _tc3_body,
        grid=(_GRID,),
        in_specs=[
            pl.BlockSpec((1, _ROWB, _H), lambda i: (0, i, 0)),
            pl.BlockSpec((1, _ROWB, _H), lambda i: (1, i, 0)),
            pl.BlockSpec((_ROWB, _H), lambda i: (i, 0)),
            pl.BlockSpec((_H, _DIN), lambda i: (0, 0)),
            vec, vec, vec,
            pl.BlockSpec((_DIN, _DIN), lambda i: (0, 0)),
            vec,
        ],
        out_specs=pl.BlockSpec((_ROWB, _DIN), lambda i: (i, 0)),
        out_shape=jax.ShapeDtypeStruct((_N, _DIN), jnp.float32),
    )(parts, parts, h1, w2a, b2a.reshape(1, _DIN), g2.reshape(1, _DIN),
      be2.reshape(1, _DIN), w2b, b2b.reshape(1, _DIN))


def _prep_edges(edge_index, edge_weight):
    pad = _NW * _EPW - _E
    src = jnp.concatenate([edge_index[0], jnp.zeros((pad,), jnp.int32)])
    dst = jnp.concatenate([edge_index[1], jnp.zeros((pad,), jnp.int32)])
    ew = jnp.concatenate([edge_weight, jnp.zeros((pad,), jnp.float32)])
    return (src.reshape(_NW, _CH, _C), dst.reshape(_NW, _CH, _C),
            ew.reshape(_NW, _CH, _C))


def kernel(x, edge_index, edge_weight, w1a, b1a, g1, be1, w1b, b1b,
           w2a, b2a, g2, be2, w2b, b2b):
    src, dst, ew = _prep_edges(edge_index, edge_weight)
    y1 = _tc1(x, w1a)
    parts1 = _segsum(y1, src, dst, ew)
    h1 = _tc2(parts1, y1, b1a, g1, be1, w1b, b1b)
    parts2 = _segsum(h1, src, dst, ew)
    return _tc3(parts2, h1, w2a, b2a, g2, be2, w2b, b2b)


# speedup vs baseline: 10.3913x; 1.0092x over previous
import functools

import jax
import jax.numpy as jnp
import numpy as np
from jax import lax
from jax.experimental import pallas as pl
from jax.experimental.pallas import tpu as pltpu
from jax.experimental.pallas import tpu_sc as plsc

_N = 10000
_E = 320000
_DIN = 128
_H = 32
_NC = 2
_NS = 16
_NW = _NC * _NS
_C = 128
_EPW = 10240
_CH = _EPW // _C
_NPAD = 10240
_RPT = _NPAD // _NS
_BN = float(1.0 / np.sqrt(1.0 + 1e-5))

_ROWB = 1000
_GRID = _N // _ROWB


def _segsum_body(y_hbm, src_hbm, dst_hbm, ew_hbm, out_hbm,
                 acc_sh, src_v, dst_v, ew_v, grow0, grow1, srow0, srow1,
                 zbuf, gsem0, gsem1, ssem0, ssem1):
    cid = lax.axis_index("c")
    sid = lax.axis_index("s")
    wid = sid * _NC + cid

    z16 = jnp.zeros((16,), jnp.float32)

    def _zero_row(i, carry):
        zbuf[i, pl.ds(0, 16)] = z16
        zbuf[i, pl.ds(16, 16)] = z16
        return carry

    lax.fori_loop(0, _RPT, _zero_row, 0)
    pltpu.sync_copy(zbuf, acc_sh.at[pl.ds(sid * _RPT, _RPT)])

    pltpu.sync_copy(src_hbm.at[wid], src_v)
    pltpu.sync_copy(dst_hbm.at[wid], dst_v)
    pltpu.sync_copy(ew_hbm.at[wid], ew_v)

    plsc.subcore_barrier()

    gbufs = ((grow0, gsem0), (grow1, gsem1))
    sbufs = ((srow0, ssem0), (srow1, ssem1))

    def _g_start(j, b):
        rows, sem = gbufs[b]
        pltpu.async_copy(y_hbm.at[src_v.at[j]], rows, sem)

    def _g_wait(j, b):
        rows, sem = gbufs[b]
        pltpu.make_async_copy(y_hbm.at[src_v.at[j]], rows, sem).wait()

    def _s_start(j, b):
        rows, sem = sbufs[b]
        pltpu.async_copy(rows, acc_sh.at[dst_v.at[j]], sem, add=True)

    def _s_wait(j, b):
        rows, sem = sbufs[b]
        pltpu.make_async_copy(rows, acc_sh.at[dst_v.at[j]], sem).wait()

    def _mul(j, b):
        grow, _ = gbufs[b]
        srow, _ = sbufs[b]

        def _scale(i, carry):
            e0 = i * 16
            wv = ew_v[j, pl.ds(e0, 16)]
            for k in range(16):
                w = wv[k]
                srow[e0 + k, pl.ds(0, 16)] = grow[e0 + k, pl.ds(0, 16)] * w
                srow[e0 + k, pl.ds(16, 16)] = grow[e0 + k, pl.ds(16, 16)] * w
            return carry

        lax.fori_loop(0, _C // 16, _scale, 0)

    _g_start(0, 0)
    _g_start(1, 1)
    for b in range(2):
        _g_wait(b, b)
        _mul(b, b)
        _g_start(b + 2, b)
        _s_start(b, b)

    def _steady(j, carry):
        for b in range(2):
            _g_wait(j + b, b)
            _s_wait(j + b - 2, b)
            _mul(j + b, b)
            _g_start(j + b + 2, b)
            _s_start(j + b, b)
        return carry

    lax.fori_loop(1, (_CH - 2) // 2, lambda g, c: _steady(g * 2, c), 0)
    for b in range(2):
        _g_wait(_CH - 2 + b, b)
        _s_wait(_CH - 4 + b, b)
        _mul(_CH - 2 + b, b)
        _s_start(_CH - 2 + b, b)
    for b in range(2):
        _s_wait(_CH - 2 + b, b)

    plsc.subcore_barrier()
    pltpu.sync_copy(
        acc_sh.at[pl.ds(sid * _RPT, _RPT)],
        out_hbm.at[cid, pl.ds(sid * _RPT, _RPT)],
    )


@functools.lru_cache(maxsize=1)
def _build_segsum():
    mesh = plsc.VectorSubcoreMesh(
        core_axis_name="c", subcore_axis_name="s",
        num_cores=_NC, num_subcores=_NS,
    )
    return pl.kernel(
        _segsum_body,
        out_type=jax.ShapeDtypeStruct((_NC, _NPAD, _H), jnp.float32),
        mesh=mesh,
        scratch_types=[
            pltpu.VMEM_SHARED((_NPAD, _H), jnp.float32),
            pltpu.VMEM((_CH, _C), jnp.int32),
            pltpu.VMEM((_CH, _C), jnp.int32),
            pltpu.VMEM((_CH, _C), jnp.float32),
            pltpu.VMEM((_C, _H), jnp.float32),
            pltpu.VMEM((_C, _H), jnp.float32),
            pltpu.VMEM((_C, _H), jnp.float32),
            pltpu.VMEM((_C, _H), jnp.float32),
            pltpu.VMEM((_RPT, _H), jnp.float32),
            pltpu.SemaphoreType.DMA,
            pltpu.SemaphoreType.DMA,
            pltpu.SemaphoreType.DMA,
            pltpu.SemaphoreType.DMA,
        ],
        compiler_params=pltpu.CompilerParams(use_tc_tiling_on_sc=False),
    )


def _segsum(y, src, dst, ew):
    return _build_segsum()(y, src, dst, ew)


def _tc1_body(x_ref, w_ref, o_ref):
    o_ref[...] = jnp.dot(x_ref[...], w_ref[...],
                         preferred_element_type=jnp.float32)


def _tc1(x, w1a):
    return pl.pallas_call(
        _tc1_body,
        grid=(_GRID,),
        in_specs=[
            pl.BlockSpec((_ROWB, _DIN), lambda i: (i, 0)),
            pl.BlockSpec((_DIN, _H), lambda i: (0, 0)),
        ],
        out_specs=pl.BlockSpec((_ROWB, _H), lambda i: (i, 0)),
        out_shape=jax.ShapeDtypeStruct((_N, _H), jnp.float32),
    )(x, w1a)


def _tc2_body(p_ref, q_ref, y_ref, b1a_ref, g1_ref, be1_ref, w1b_ref,
              b1b_ref, o_ref):
    t = p_ref[0] + q_ref[0] + y_ref[...] + b1a_ref[...]
    t = t * (g1_ref[...] * _BN) + be1_ref[...]
    t = jnp.maximum(t, 0.0)
    t = jnp.dot(t, w1b_ref[...], preferred_element_type=jnp.float32)
    o_ref[...] = jnp.maximum(t + b1b_ref[...], 0.0)


def _tc2(parts, y1, b1a, g1, be1, w1b, b1b):
    vec = pl.BlockSpec((1, _H), lambda i: (0, 0))
    return pl.pallas_call(
        _tc2_body,
        grid=(_GRID,),
        in_specs=[
            pl.BlockSpec((1, _ROWB, _H), lambda i: (0, i, 0)),
            pl.BlockSpec((1, _ROWB, _H), lambda i: (1, i, 0)),
            pl.BlockSpec((_ROWB, _H), lambda i: (i, 0)),
            vec, vec, vec,
            pl.BlockSpec((_H, _H), lambda i: (0, 0)),
            vec,
        ],
        out_specs=pl.BlockSpec((_ROWB, _H), lambda i: (i, 0)),
        out_shape=jax.ShapeDtypeStruct((_N, _H), jnp.float32),
    )(parts, parts, y1, b1a.reshape(1, _H), g1.reshape(1, _H),
      be1.reshape(1, _H), w1b, b1b.reshape(1, _H))


def _tc3_body(p_ref, q_ref, h_ref, w2a_ref, b2a_ref, g2_ref, be2_ref,
              w2b_ref, b2b_ref, o_ref):
    t = p_ref[0] + q_ref[0] + h_ref[...]
    t = jnp.dot(t, w2a_ref[...], preferred_element_type=jnp.float32)
    t = (t + b2a_ref[...]) * (g2_ref[...] * _BN) + be2_ref[...]
    t = jnp.maximum(t, 0.0)
    t = jnp.dot(t, w2b_ref[...], preferred_element_type=jnp.float32)
    o_ref[...] = t + b2b_ref[...]


def _tc3(parts, h1, w2a, b2a, g2, be2, w2b, b2b):
    vec = pl.BlockSpec((1, _DIN), lambda i: (0, 0))
    return pl.pallas_call(
        _tc3_body,
        grid=(_GRID,),
        in_specs=[
            pl.BlockSpec((1, _ROWB, _H), lambda i: (0, i, 0)),
            pl.BlockSpec((1, _ROWB, _H), lambda i: (1, i, 0)),
            pl.BlockSpec((_ROWB, _H), lambda i: (i, 0)),
            pl.BlockSpec((_H, _DIN), lambda i: (0, 0)),
            vec, vec, vec,
            pl.BlockSpec((_DIN, _DIN), lambda i: (0, 0)),
            vec,
        ],
        out_specs=pl.BlockSpec((_ROWB, _DIN), lambda i: (i, 0)),
        out_shape=jax.ShapeDtypeStruct((_N, _DIN), jnp.float32),
    )(parts, parts, h1, w2a, b2a.reshape(1, _DIN), g2.reshape(1, _DIN),
      be2.reshape(1, _DIN), w2b, b2b.reshape(1, _DIN))


def _prep_edges(edge_index, edge_weight):
    pad = _NW * _EPW - _E
    src = jnp.concatenate([edge_index[0], jnp.zeros((pad,), jnp.int32)])
    dst = jnp.concatenate([edge_index[1], jnp.zeros((pad,), jnp.int32)])
    ew = jnp.concatenate([edge_weight, jnp.zeros((pad,), jnp.float32)])
    return (src.reshape(_NW, _CH, _C), dst.reshape(_NW, _CH, _C),
            ew.reshape(_NW, _CH, _C))


def kernel(x, edge_index, edge_weight, w1a, b1a, g1, be1, w1b, b1b,
           w2a, b2a, g2, be2, w2b, b2b):
    src, dst, ew = _prep_edges(edge_index, edge_weight)
    y1 = _tc1(x, w1a)
    parts1 = _segsum(y1, src, dst, ew)
    h1 = _tc2(parts1, y1, b1a, g1, be1, w1b, b1b)
    parts2 = _segsum(h1, src, dst, ew)
    return _tc3(parts2, h1, w2a, b2a, g2, be2, w2b, b2b)


# speedup vs baseline: 12.9553x; 1.2467x over previous
import functools

import jax
import jax.numpy as jnp
import numpy as np
from jax import lax
from jax.experimental import pallas as pl
from jax.experimental.pallas import tpu as pltpu
from jax.experimental.pallas import tpu_sc as plsc

_N = 10000
_E = 320000
_DIN = 128
_H = 32
_NC = 2
_NS = 16
_NW = _NC * _NS
_C = 128
_EPW = 10240
_CH = _EPW // _C
_NPAD = 10240
_RPT = _NPAD // _NS
_BN = float(1.0 / np.sqrt(1.0 + 1e-5))
_U = np.concatenate([np.arange(0, 32, 2), np.arange(1, 32, 2)])

_ROWB = 1000
_GRID = _N // _ROWB


def _segsum_body(y_hbm, src_hbm, dst_hbm, ew_hbm, out_hbm,
                 acc_sh, src_v, dst_v, ew_v, grow0, grow1, srow0, srow1,
                 zbuf, gsem0, gsem1, ssem0, ssem1):
    cid = lax.axis_index("c")
    sid = lax.axis_index("s")
    wid = sid * _NC + cid

    z16 = jnp.zeros((16,), jnp.float32)

    def _zero_row(i, carry):
        zbuf[i, pl.ds(0, 16)] = z16
        zbuf[i, pl.ds(16, 16)] = z16
        return carry

    lax.fori_loop(0, _RPT, _zero_row, 0)
    pltpu.sync_copy(zbuf, acc_sh.at[pl.ds(sid * _RPT, _RPT)])

    pltpu.sync_copy(src_hbm.at[wid], src_v)
    pltpu.sync_copy(dst_hbm.at[wid], dst_v)
    pltpu.sync_copy(ew_hbm.at[wid], ew_v)

    plsc.subcore_barrier()

    gbufs = ((grow0, gsem0), (grow1, gsem1))
    sbufs = ((srow0, ssem0), (srow1, ssem1))

    def _g_start(j, b):
        rows, sem = gbufs[b]
        pltpu.async_copy(y_hbm.at[src_v.at[j]], rows, sem)

    def _g_wait(j, b):
        rows, sem = gbufs[b]
        pltpu.make_async_copy(y_hbm.at[src_v.at[j]], rows, sem).wait()

    def _s_start(j, b):
        rows, sem = sbufs[b]
        pltpu.async_copy(rows, acc_sh.at[dst_v.at[j]], sem, add=True)

    def _s_wait(j, b):
        rows, sem = sbufs[b]
        pltpu.make_async_copy(rows, acc_sh.at[dst_v.at[j]], sem).wait()

    def _mul(j, b):
        grow, _ = gbufs[b]
        srow, _ = sbufs[b]

        def _scale(i, carry):
            e0 = i * 16
            wv = ew_v[j, pl.ds(e0, 16)]
            for k in range(16):
                w = wv[k]
                a, b2 = plsc.unpack(grow[e0 + k, :],
                                    format=plsc.PackFormat.INTERLEAVED)
                srow[e0 + k, pl.ds(0, 16)] = a * w
                srow[e0 + k, pl.ds(16, 16)] = b2 * w
            return carry

        lax.fori_loop(0, _C // 16, _scale, 0)

    _g_start(0, 0)
    _g_start(1, 1)
    for b in range(2):
        _g_wait(b, b)
        _mul(b, b)
        _g_start(b + 2, b)
        _s_start(b, b)

    def _steady(j, carry):
        for b in range(2):
            _g_wait(j + b, b)
            _s_wait(j + b - 2, b)
            _mul(j + b, b)
            _g_start(j + b + 2, b)
            _s_start(j + b, b)
        return carry

    lax.fori_loop(1, (_CH - 2) // 2, lambda g, c: _steady(g * 2, c), 0)
    for b in range(2):
        _g_wait(_CH - 2 + b, b)
        _s_wait(_CH - 4 + b, b)
        _mul(_CH - 2 + b, b)
        _s_start(_CH - 2 + b, b)
    for b in range(2):
        _s_wait(_CH - 2 + b, b)

    plsc.subcore_barrier()
    pltpu.sync_copy(
        acc_sh.at[pl.ds(sid * _RPT, _RPT)],
        out_hbm.at[cid, pl.ds(sid * _RPT, _RPT)],
    )


@functools.lru_cache(maxsize=1)
def _build_segsum():
    mesh = plsc.VectorSubcoreMesh(
        core_axis_name="c", subcore_axis_name="s",
        num_cores=_NC, num_subcores=_NS,
    )
    return pl.kernel(
        _segsum_body,
        out_type=jax.ShapeDtypeStruct((_NC, _NPAD, _H), jnp.float32),
        mesh=mesh,
        scratch_types=[
            pltpu.VMEM_SHARED((_NPAD, _H), jnp.float32),
            pltpu.VMEM((_CH, _C), jnp.int32),
            pltpu.VMEM((_CH, _C), jnp.int32),
            pltpu.VMEM((_CH, _C), jnp.float32),
            pltpu.VMEM((_C, _H), jnp.bfloat16),
            pltpu.VMEM((_C, _H), jnp.bfloat16),
            pltpu.VMEM((_C, _H), jnp.float32),
            pltpu.VMEM((_C, _H), jnp.float32),
            pltpu.VMEM((_RPT, _H), jnp.float32),
            pltpu.SemaphoreType.DMA,
            pltpu.SemaphoreType.DMA,
            pltpu.SemaphoreType.DMA,
            pltpu.SemaphoreType.DMA,
        ],
        compiler_params=pltpu.CompilerParams(
            use_tc_tiling_on_sc=False, needs_layout_passes=False),
    )


def _segsum(y, src, dst, ew):
    return _build_segsum()(y, src, dst, ew)


def _tc1_body(x_ref, wU_ref, w_ref, oU_ref, obf_ref):
    x = x_ref[...]
    oU_ref[...] = jnp.dot(x, wU_ref[...], preferred_element_type=jnp.float32)
    obf_ref[...] = jnp.dot(x, w_ref[...],
                           preferred_element_type=jnp.float32
                           ).astype(jnp.bfloat16)


def _tc1(x, w1aU, w1a):
    return pl.pallas_call(
        _tc1_body,
        grid=(_GRID,),
        in_specs=[
            pl.BlockSpec((_ROWB, _DIN), lambda i: (i, 0)),
            pl.BlockSpec((_DIN, _H), lambda i: (0, 0)),
            pl.BlockSpec((_DIN, _H), lambda i: (0, 0)),
        ],
        out_specs=[
            pl.BlockSpec((_ROWB, _H), lambda i: (i, 0)),
            pl.BlockSpec((_ROWB, _H), lambda i: (i, 0)),
        ],
        out_shape=[
            jax.ShapeDtypeStruct((_N, _H), jnp.float32),
            jax.ShapeDtypeStruct((_N, _H), jnp.bfloat16),
        ],
    )(x, w1aU, w1a)


def _tc2_body(p_ref, q_ref, y_ref, b1a_ref, g1_ref, be1_ref, w1bU_ref,
              w1b_ref, b1bU_ref, b1b_ref, oU_ref, obf_ref):
    t = p_ref[0] + q_ref[0] + y_ref[...] + b1a_ref[...]
    t = t * (g1_ref[...] * _BN) + be1_ref[...]
    t = jnp.maximum(t, 0.0)
    hU = jnp.dot(t, w1bU_ref[...], preferred_element_type=jnp.float32)
    oU_ref[...] = jnp.maximum(hU + b1bU_ref[...], 0.0)
    h = jnp.dot(t, w1b_ref[...], preferred_element_type=jnp.float32)
    obf_ref[...] = jnp.maximum(h + b1b_ref[...], 0.0).astype(jnp.bfloat16)


def _tc2(parts, y1U, b1aU, g1U, be1U, w1bUU, w1bU, b1bU, b1b):
    vec = pl.BlockSpec((1, _H), lambda i: (0, 0))
    mat = pl.BlockSpec((_H, _H), lambda i: (0, 0))
    return pl.pallas_call(
        _tc2_body,
        grid=(_GRID,),
        in_specs=[
            pl.BlockSpec((1, _ROWB, _H), lambda i: (0, i, 0)),
            pl.BlockSpec((1, _ROWB, _H), lambda i: (1, i, 0)),
            pl.BlockSpec((_ROWB, _H), lambda i: (i, 0)),
            vec, vec, vec, mat, mat, vec, vec,
        ],
        out_specs=[
            pl.BlockSpec((_ROWB, _H), lambda i: (i, 0)),
            pl.BlockSpec((_ROWB, _H), lambda i: (i, 0)),
        ],
        out_shape=[
            jax.ShapeDtypeStruct((_N, _H), jnp.float32),
            jax.ShapeDtypeStruct((_N, _H), jnp.bfloat16),
        ],
    )(parts, parts, y1U, b1aU.reshape(1, _H), g1U.reshape(1, _H),
      be1U.reshape(1, _H), w1bUU, w1bU, b1bU.reshape(1, _H),
      b1b.reshape(1, _H))


def _tc3_body(p_ref, q_ref, h_ref, w2a_ref, b2a_ref, g2_ref, be2_ref,
              w2b_ref, b2b_ref, o_ref):
    t = p_ref[0] + q_ref[0] + h_ref[...]
    t = jnp.dot(t, w2a_ref[...], preferred_element_type=jnp.float32)
    t = (t + b2a_ref[...]) * (g2_ref[...] * _BN) + be2_ref[...]
    t = jnp.maximum(t, 0.0)
    t = jnp.dot(t, w2b_ref[...], preferred_element_type=jnp.float32)
    o_ref[...] = t + b2b_ref[...]


def _tc3(parts, h1U, w2aU, b2a, g2, be2, w2b, b2b):
    vec = pl.BlockSpec((1, _DIN), lambda i: (0, 0))
    return pl.pallas_call(
        _tc3_body,
        grid=(_GRID,),
        in_specs=[
            pl.BlockSpec((1, _ROWB, _H), lambda i: (0, i, 0)),
            pl.BlockSpec((1, _ROWB, _H), lambda i: (1, i, 0)),
            pl.BlockSpec((_ROWB, _H), lambda i: (i, 0)),
            pl.BlockSpec((_H, _DIN), lambda i: (0, 0)),
            vec, vec, vec,
            pl.BlockSpec((_DIN, _DIN), lambda i: (0, 0)),
            vec,
        ],
        out_specs=pl.BlockSpec((_ROWB, _DIN), lambda i: (i, 0)),
        out_shape=jax.ShapeDtypeStruct((_N, _DIN), jnp.float32),
    )(parts, parts, h1U, w2aU, b2a.reshape(1, _DIN), g2.reshape(1, _DIN),
      be2.reshape(1, _DIN), w2b, b2b.reshape(1, _DIN))


def _prep_edges(edge_index, edge_weight):
    pad = _NW * _EPW - _E
    src = jnp.concatenate([edge_index[0], jnp.zeros((pad,), jnp.int32)])
    dst = jnp.concatenate([edge_index[1], jnp.zeros((pad,), jnp.int32)])
    ew = jnp.concatenate([edge_weight, jnp.zeros((pad,), jnp.float32)])
    return (src.reshape(_NW, _CH, _C), dst.reshape(_NW, _CH, _C),
            ew.reshape(_NW, _CH, _C))


def kernel(x, edge_index, edge_weight, w1a, b1a, g1, be1, w1b, b1b,
           w2a, b2a, g2, be2, w2b, b2b):
    src, dst, ew = _prep_edges(edge_index, edge_weight)
    w1aU = w1a[:, _U]
    w1bU = w1b[_U, :]
    y1U, y1bf = _tc1(x, w1aU, w1a)
    parts1 = _segsum(y1bf, src, dst, ew)
    h1U, h1bf = _tc2(parts1, y1U, b1a[_U], g1[_U], be1[_U],
                     w1bU[:, _U], w1bU, b1b[_U], b1b)
    parts2 = _segsum(h1bf, src, dst, ew)
    return _tc3(parts2, h1U, w2a[_U, :], b2a, g2, be2, w2b, b2b)


# speedup vs baseline: 14.3307x; 1.1062x over previous
import functools

import jax
import jax.numpy as jnp
import numpy as np
from jax import lax
from jax.experimental import pallas as pl
from jax.experimental.pallas import tpu as pltpu
from jax.experimental.pallas import tpu_sc as plsc

_N = 10000
_E = 320000
_DIN = 128
_H = 32
_NC = 2
_NS = 16
_NW = _NC * _NS
_C = 128
_EPW = 10240
_CH = _EPW // _C
_NPAD = 10240
_RPT = _NPAD // _NS
_BN = float(1.0 / np.sqrt(1.0 + 1e-5))
_U = np.concatenate([np.arange(0, 32, 2), np.arange(1, 32, 2)])

_ROWB = 1000
_GRID = _N // _ROWB


def _segsum_body(y_hbm, src_hbm, dst_hbm, ew_hbm, out_hbm,
                 acc_sh, src_v, dst_v, ew_v, grow0, grow1, srow0, srow1,
                 zbuf, gsem0, gsem1, ssem0, ssem1):
    cid = lax.axis_index("c")
    sid = lax.axis_index("s")
    wid = sid * _NC + cid

    z16 = jnp.zeros((16,), jnp.float32)

    def _zero_row(i, carry):
        zbuf[i, pl.ds(0, 16)] = z16
        zbuf[i, pl.ds(16, 16)] = z16
        return carry

    lax.fori_loop(0, _RPT, _zero_row, 0)
    pltpu.sync_copy(zbuf, acc_sh.at[pl.ds(sid * _RPT, _RPT)])

    pltpu.sync_copy(src_hbm.at[wid], src_v)
    pltpu.sync_copy(dst_hbm.at[wid], dst_v)
    pltpu.sync_copy(ew_hbm.at[wid], ew_v)

    plsc.subcore_barrier()

    gbufs = ((grow0, gsem0), (grow1, gsem1))
    sbufs = ((srow0, ssem0), (srow1, ssem1))

    def _g_start(j, b):
        rows, sem = gbufs[b]
        pltpu.async_copy(y_hbm.at[src_v.at[j]], rows, sem)

    def _g_wait(j, b):
        rows, sem = gbufs[b]
        pltpu.make_async_copy(y_hbm.at[src_v.at[j]], rows, sem).wait()

    def _s_start(j, b):
        rows, sem = sbufs[b]
        pltpu.async_copy(rows, acc_sh.at[dst_v.at[j]], sem, add=True)

    def _s_wait(j, b):
        rows, sem = sbufs[b]
        pltpu.make_async_copy(rows, acc_sh.at[dst_v.at[j]], sem).wait()

    def _mul(j, b):
        grow, _ = gbufs[b]
        srow, _ = sbufs[b]

        @plsc.parallel_loop(0, _C // 16, unroll=2)
        def _scale(i):
            e0 = i * 16
            wv = ew_v[j, pl.ds(e0, 16)]
            for k in range(16):
                w = wv[k]
                a, b2 = plsc.unpack(grow[e0 + k, :],
                                    format=plsc.PackFormat.INTERLEAVED)
                srow[e0 + k, pl.ds(0, 16)] = a * w
                srow[e0 + k, pl.ds(16, 16)] = b2 * w

    _g_start(0, 0)
    _g_start(1, 1)
    for b in range(2):
        _g_wait(b, b)
        _mul(b, b)
        _g_start(b + 2, b)
        _s_start(b, b)

    def _steady(j, carry):
        for b in range(2):
            _g_wait(j + b, b)
            _s_wait(j + b - 2, b)
            _mul(j + b, b)
            _g_start(j + b + 2, b)
            _s_start(j + b, b)
        return carry

    lax.fori_loop(1, (_CH - 2) // 2, lambda g, c: _steady(g * 2, c), 0)
    for b in range(2):
        _g_wait(_CH - 2 + b, b)
        _s_wait(_CH - 4 + b, b)
        _mul(_CH - 2 + b, b)
        _s_start(_CH - 2 + b, b)
    for b in range(2):
        _s_wait(_CH - 2 + b, b)

    plsc.subcore_barrier()
    pltpu.sync_copy(
        acc_sh.at[pl.ds(sid * _RPT, _RPT)],
        out_hbm.at[cid, pl.ds(sid * _RPT, _RPT)],
    )


@functools.lru_cache(maxsize=1)
def _build_segsum():
    mesh = plsc.VectorSubcoreMesh(
        core_axis_name="c", subcore_axis_name="s",
        num_cores=_NC, num_subcores=_NS,
    )
    return pl.kernel(
        _segsum_body,
        out_type=jax.ShapeDtypeStruct((_NC, _NPAD, _H), jnp.float32),
        mesh=mesh,
        scratch_types=[
            pltpu.VMEM_SHARED((_NPAD, _H), jnp.float32),
            pltpu.VMEM((_CH, _C), jnp.int32),
            pltpu.VMEM((_CH, _C), jnp.int32),
            pltpu.VMEM((_CH, _C), jnp.float32),
            pltpu.VMEM((_C, _H), jnp.bfloat16),
            pltpu.VMEM((_C, _H), jnp.bfloat16),
            pltpu.VMEM((_C, _H), jnp.float32),
            pltpu.VMEM((_C, _H), jnp.float32),
            pltpu.VMEM((_RPT, _H), jnp.float32),
            pltpu.SemaphoreType.DMA,
            pltpu.SemaphoreType.DMA,
            pltpu.SemaphoreType.DMA,
            pltpu.SemaphoreType.DMA,
        ],
        compiler_params=pltpu.CompilerParams(
            use_tc_tiling_on_sc=False, needs_layout_passes=False),
    )


def _segsum(y, src, dst, ew):
    return _build_segsum()(y, src, dst, ew)


def _tc1_body(x_ref, wU_ref, w_ref, oU_ref, obf_ref):
    x = x_ref[...]
    oU_ref[...] = jnp.dot(x, wU_ref[...], preferred_element_type=jnp.float32)
    obf_ref[...] = jnp.dot(x, w_ref[...],
                           preferred_element_type=jnp.float32
                           ).astype(jnp.bfloat16)


def _tc1(x, w1aU, w1a):
    return pl.pallas_call(
        _tc1_body,
        grid=(_GRID,),
        in_specs=[
            pl.BlockSpec((_ROWB, _DIN), lambda i: (i, 0)),
            pl.BlockSpec((_DIN, _H), lambda i: (0, 0)),
            pl.BlockSpec((_DIN, _H), lambda i: (0, 0)),
        ],
        out_specs=[
            pl.BlockSpec((_ROWB, _H), lambda i: (i, 0)),
            pl.BlockSpec((_ROWB, _H), lambda i: (i, 0)),
        ],
        out_shape=[
            jax.ShapeDtypeStruct((_N, _H), jnp.float32),
            jax.ShapeDtypeStruct((_N, _H), jnp.bfloat16),
        ],
    )(x, w1aU, w1a)


def _tc2_body(p_ref, q_ref, y_ref, b1a_ref, g1_ref, be1_ref, w1bU_ref,
              w1b_ref, b1bU_ref, b1b_ref, oU_ref, obf_ref):
    t = p_ref[0] + q_ref[0] + y_ref[...] + b1a_ref[...]
    t = t * (g1_ref[...] * _BN) + be1_ref[...]
    t = jnp.maximum(t, 0.0)
    hU = jnp.dot(t, w1bU_ref[...], preferred_element_type=jnp.float32)
    oU_ref[...] = jnp.maximum(hU + b1bU_ref[...], 0.0)
    h = jnp.dot(t, w1b_ref[...], preferred_element_type=jnp.float32)
    obf_ref[...] = jnp.maximum(h + b1b_ref[...], 0.0).astype(jnp.bfloat16)


def _tc2(parts, y1U, b1aU, g1U, be1U, w1bUU, w1bU, b1bU, b1b):
    vec = pl.BlockSpec((1, _H), lambda i: (0, 0))
    mat = pl.BlockSpec((_H, _H), lambda i: (0, 0))
    return pl.pallas_call(
        _tc2_body,
        grid=(_GRID,),
        in_specs=[
            pl.BlockSpec((1, _ROWB, _H), lambda i: (0, i, 0)),
            pl.BlockSpec((1, _ROWB, _H), lambda i: (1, i, 0)),
            pl.BlockSpec((_ROWB, _H), lambda i: (i, 0)),
            vec, vec, vec, mat, mat, vec, vec,
        ],
        out_specs=[
            pl.BlockSpec((_ROWB, _H), lambda i: (i, 0)),
            pl.BlockSpec((_ROWB, _H), lambda i: (i, 0)),
        ],
        out_shape=[
            jax.ShapeDtypeStruct((_N, _H), jnp.float32),
            jax.ShapeDtypeStruct((_N, _H), jnp.bfloat16),
        ],
    )(parts, parts, y1U, b1aU.reshape(1, _H), g1U.reshape(1, _H),
      be1U.reshape(1, _H), w1bUU, w1bU, b1bU.reshape(1, _H),
      b1b.reshape(1, _H))


def _tc3_body(p_ref, q_ref, h_ref, w2a_ref, b2a_ref, g2_ref, be2_ref,
              w2b_ref, b2b_ref, o_ref):
    t = p_ref[0] + q_ref[0] + h_ref[...]
    t = jnp.dot(t, w2a_ref[...], preferred_element_type=jnp.float32)
    t = (t + b2a_ref[...]) * (g2_ref[...] * _BN) + be2_ref[...]
    t = jnp.maximum(t, 0.0)
    t = jnp.dot(t, w2b_ref[...], preferred_element_type=jnp.float32)
    o_ref[...] = t + b2b_ref[...]


def _tc3(parts, h1U, w2aU, b2a, g2, be2, w2b, b2b):
    vec = pl.BlockSpec((1, _DIN), lambda i: (0, 0))
    return pl.pallas_call(
        _tc3_body,
        grid=(_GRID,),
        in_specs=[
            pl.BlockSpec((1, _ROWB, _H), lambda i: (0, i, 0)),
            pl.BlockSpec((1, _ROWB, _H), lambda i: (1, i, 0)),
            pl.BlockSpec((_ROWB, _H), lambda i: (i, 0)),
            pl.BlockSpec((_H, _DIN), lambda i: (0, 0)),
            vec, vec, vec,
            pl.BlockSpec((_DIN, _DIN), lambda i: (0, 0)),
            vec,
        ],
        out_specs=pl.BlockSpec((_ROWB, _DIN), lambda i: (i, 0)),
        out_shape=jax.ShapeDtypeStruct((_N, _DIN), jnp.float32),
    )(parts, parts, h1U, w2aU, b2a.reshape(1, _DIN), g2.reshape(1, _DIN),
      be2.reshape(1, _DIN), w2b, b2b.reshape(1, _DIN))


def _prep_edges(edge_index, edge_weight):
    pad = _NW * _EPW - _E
    src = jnp.concatenate([edge_index[0], jnp.zeros((pad,), jnp.int32)])
    dst = jnp.concatenate([edge_index[1], jnp.zeros((pad,), jnp.int32)])
    ew = jnp.concatenate([edge_weight, jnp.zeros((pad,), jnp.float32)])
    return (src.reshape(_NW, _CH, _C), dst.reshape(_NW, _CH, _C),
            ew.reshape(_NW, _CH, _C))


def kernel(x, edge_index, edge_weight, w1a, b1a, g1, be1, w1b, b1b,
           w2a, b2a, g2, be2, w2b, b2b):
    src, dst, ew = _prep_edges(edge_index, edge_weight)
    w1aU = w1a[:, _U]
    w1bU = w1b[_U, :]
    y1U, y1bf = _tc1(x, w1aU, w1a)
    parts1 = _segsum(y1bf, src, dst, ew)
    h1U, h1bf = _tc2(parts1, y1U, b1a[_U], g1[_U], be1[_U],
                     w1bU[:, _U], w1bU, b1b[_U], b1b)
    parts2 = _segsum(h1bf, src, dst, ew)
    return _tc3(parts2, h1U, w2a[_U, :], b2a, g2, be2, w2b, b2b)


# speedup vs baseline: 14.5006x; 1.0119x over previous
import functools

import jax
import jax.numpy as jnp
import numpy as np
from jax import lax
from jax.experimental import pallas as pl
from jax.experimental.pallas import tpu as pltpu
from jax.experimental.pallas import tpu_sc as plsc

_N = 10000
_E = 320000
_DIN = 128
_H = 32
_NC = 2
_NS = 16
_NW = _NC * _NS
_C = 128
_EPW = 10240
_CH = _EPW // _C
_NPAD = 10240
_RPT = _NPAD // _NS
_BN = float(1.0 / np.sqrt(1.0 + 1e-5))
_U = np.concatenate([np.arange(0, 32, 2), np.arange(1, 32, 2)])

_ROWB = 1000
_GRID = _N // _ROWB


def _segsum_body(y_hbm, src_hbm, dst_hbm, ew_hbm, out_hbm,
                 acc_sh, src_v, dst_v, ew_v, grow0, grow1, srow0, srow1,
                 zbuf, gsem0, gsem1, ssem0, ssem1):
    cid = lax.axis_index("c")
    sid = lax.axis_index("s")
    wid = sid * _NC + cid

    z16 = jnp.zeros((16,), jnp.float32)

    def _zero_row(i, carry):
        zbuf[i, pl.ds(0, 16)] = z16
        zbuf[i, pl.ds(16, 16)] = z16
        return carry

    lax.fori_loop(0, _RPT, _zero_row, 0)
    pltpu.sync_copy(zbuf, acc_sh.at[pl.ds(sid * _RPT, _RPT)])

    pltpu.sync_copy(src_hbm.at[wid], src_v)
    pltpu.sync_copy(dst_hbm.at[wid], dst_v)
    pltpu.sync_copy(ew_hbm.at[wid], ew_v)

    plsc.subcore_barrier()

    gbufs = ((grow0, gsem0), (grow1, gsem1))
    sbufs = ((srow0, ssem0), (srow1, ssem1))

    def _g_start(j, b):
        rows, sem = gbufs[b]
        pltpu.async_copy(y_hbm.at[src_v.at[j]], rows, sem)

    def _g_wait(j, b):
        rows, sem = gbufs[b]
        pltpu.make_async_copy(y_hbm.at[src_v.at[j]], rows, sem).wait()

    def _s_start(j, b):
        rows, sem = sbufs[b]
        pltpu.async_copy(rows, acc_sh.at[dst_v.at[j]], sem, add=True)

    def _s_wait(j, b):
        rows, sem = sbufs[b]
        pltpu.make_async_copy(rows, acc_sh.at[dst_v.at[j]], sem).wait()

    def _mul(j, b):
        grow, _ = gbufs[b]
        srow, _ = sbufs[b]

        @plsc.parallel_loop(0, _C // 16, unroll=4)
        def _scale(i):
            e0 = i * 16
            wv = ew_v[j, pl.ds(e0, 16)]
            for k in range(16):
                w = wv[k]
                a, b2 = plsc.unpack(grow[e0 + k, :],
                                    format=plsc.PackFormat.INTERLEAVED)
                srow[e0 + k, pl.ds(0, 16)] = a * w
                srow[e0 + k, pl.ds(16, 16)] = b2 * w

    _g_start(0, 0)
    _g_start(1, 1)
    for b in range(2):
        _g_wait(b, b)
        _mul(b, b)
        _g_start(b + 2, b)
        _s_start(b, b)

    def _steady(j, carry):
        for b in range(2):
            _g_wait(j + b, b)
            _s_wait(j + b - 2, b)
            _mul(j + b, b)
            _g_start(j + b + 2, b)
            _s_start(j + b, b)
        return carry

    lax.fori_loop(1, (_CH - 2) // 2, lambda g, c: _steady(g * 2, c), 0)
    for b in range(2):
        _g_wait(_CH - 2 + b, b)
        _s_wait(_CH - 4 + b, b)
        _mul(_CH - 2 + b, b)
        _s_start(_CH - 2 + b, b)
    for b in range(2):
        _s_wait(_CH - 2 + b, b)

    plsc.subcore_barrier()
    pltpu.sync_copy(
        acc_sh.at[pl.ds(sid * _RPT, _RPT)],
        out_hbm.at[cid, pl.ds(sid * _RPT, _RPT)],
    )


@functools.lru_cache(maxsize=1)
def _build_segsum():
    mesh = plsc.VectorSubcoreMesh(
        core_axis_name="c", subcore_axis_name="s",
        num_cores=_NC, num_subcores=_NS,
    )
    return pl.kernel(
        _segsum_body,
        out_type=jax.ShapeDtypeStruct((_NC, _NPAD, _H), jnp.float32),
        mesh=mesh,
        scratch_types=[
            pltpu.VMEM_SHARED((_NPAD, _H), jnp.float32),
            pltpu.VMEM((_CH, _C), jnp.int32),
            pltpu.VMEM((_CH, _C), jnp.int32),
            pltpu.VMEM((_CH, _C), jnp.float32),
            pltpu.VMEM((_C, _H), jnp.bfloat16),
            pltpu.VMEM((_C, _H), jnp.bfloat16),
            pltpu.VMEM((_C, _H), jnp.float32),
            pltpu.VMEM((_C, _H), jnp.float32),
            pltpu.VMEM((_RPT, _H), jnp.float32),
            pltpu.SemaphoreType.DMA,
            pltpu.SemaphoreType.DMA,
            pltpu.SemaphoreType.DMA,
            pltpu.SemaphoreType.DMA,
        ],
        compiler_params=pltpu.CompilerParams(
            use_tc_tiling_on_sc=False, needs_layout_passes=False),
    )


def _segsum(y, src, dst, ew):
    return _build_segsum()(y, src, dst, ew)


def _tc1_body(x_ref, wU_ref, w_ref, oU_ref, obf_ref):
    x = x_ref[...]
    oU_ref[...] = jnp.dot(x, wU_ref[...], preferred_element_type=jnp.float32)
    obf_ref[...] = jnp.dot(x, w_ref[...],
                           preferred_element_type=jnp.float32
                           ).astype(jnp.bfloat16)


def _tc1(x, w1aU, w1a):
    return pl.pallas_call(
        _tc1_body,
        grid=(_GRID,),
        in_specs=[
            pl.BlockSpec((_ROWB, _DIN), lambda i: (i, 0)),
            pl.BlockSpec((_DIN, _H), lambda i: (0, 0)),
            pl.BlockSpec((_DIN, _H), lambda i: (0, 0)),
        ],
        out_specs=[
            pl.BlockSpec((_ROWB, _H), lambda i: (i, 0)),
            pl.BlockSpec((_ROWB, _H), lambda i: (i, 0)),
        ],
        out_shape=[
            jax.ShapeDtypeStruct((_N, _H), jnp.float32),
            jax.ShapeDtypeStruct((_N, _H), jnp.bfloat16),
        ],
    )(x, w1aU, w1a)


def _tc2_body(p_ref, q_ref, y_ref, b1a_ref, g1_ref, be1_ref, w1bU_ref,
              w1b_ref, b1bU_ref, b1b_ref, oU_ref, obf_ref):
    t = p_ref[0] + q_ref[0] + y_ref[...] + b1a_ref[...]
    t = t * (g1_ref[...] * _BN) + be1_ref[...]
    t = jnp.maximum(t, 0.0)
    hU = jnp.dot(t, w1bU_ref[...], preferred_element_type=jnp.float32)
    oU_ref[...] = jnp.maximum(hU + b1bU_ref[...], 0.0)
    h = jnp.dot(t, w1b_ref[...], preferred_element_type=jnp.float32)
    obf_ref[...] = jnp.maximum(h + b1b_ref[...], 0.0).astype(jnp.bfloat16)


def _tc2(parts, y1U, b1aU, g1U, be1U, w1bUU, w1bU, b1bU, b1b):
    vec = pl.BlockSpec((1, _H), lambda i: (0, 0))
    mat = pl.BlockSpec((_H, _H), lambda i: (0, 0))
    return pl.pallas_call(
        _tc2_body,
        grid=(_GRID,),
        in_specs=[
            pl.BlockSpec((1, _ROWB, _H), lambda i: (0, i, 0)),
            pl.BlockSpec((1, _ROWB, _H), lambda i: (1, i, 0)),
            pl.BlockSpec((_ROWB, _H), lambda i: (i, 0)),
            vec, vec, vec, mat, mat, vec, vec,
        ],
        out_specs=[
            pl.BlockSpec((_ROWB, _H), lambda i: (i, 0)),
            pl.BlockSpec((_ROWB, _H), lambda i: (i, 0)),
        ],
        out_shape=[
            jax.ShapeDtypeStruct((_N, _H), jnp.float32),
            jax.ShapeDtypeStruct((_N, _H), jnp.bfloat16),
        ],
    )(parts, parts, y1U, b1aU.reshape(1, _H), g1U.reshape(1, _H),
      be1U.reshape(1, _H), w1bUU, w1bU, b1bU.reshape(1, _H),
      b1b.reshape(1, _H))


def _tc3_body(p_ref, q_ref, h_ref, w2a_ref, b2a_ref, g2_ref, be2_ref,
              w2b_ref, b2b_ref, o_ref):
    t = p_ref[0] + q_ref[0] + h_ref[...]
    t = jnp.dot(t, w2a_ref[...], preferred_element_type=jnp.float32)
    t = (t + b2a_ref[...]) * (g2_ref[...] * _BN) + be2_ref[...]
    t = jnp.maximum(t, 0.0)
    t = jnp.dot(t, w2b_ref[...], preferred_element_type=jnp.float32)
    o_ref[...] = t + b2b_ref[...]


def _tc3(parts, h1U, w2aU, b2a, g2, be2, w2b, b2b):
    vec = pl.BlockSpec((1, _DIN), lambda i: (0, 0))
    return pl.pallas_call(
        _tc3_body,
        grid=(_GRID,),
        in_specs=[
            pl.BlockSpec((1, _ROWB, _H), lambda i: (0, i, 0)),
            pl.BlockSpec((1, _ROWB, _H), lambda i: (1, i, 0)),
            pl.BlockSpec((_ROWB, _H), lambda i: (i, 0)),
            pl.BlockSpec((_H, _DIN), lambda i: (0, 0)),
            vec, vec, vec,
            pl.BlockSpec((_DIN, _DIN), lambda i: (0, 0)),
            vec,
        ],
        out_specs=pl.BlockSpec((_ROWB, _DIN), lambda i: (i, 0)),
        out_shape=jax.ShapeDtypeStruct((_N, _DIN), jnp.float32),
    )(parts, parts, h1U, w2aU, b2a.reshape(1, _DIN), g2.reshape(1, _DIN),
      be2.reshape(1, _DIN), w2b, b2b.reshape(1, _DIN))


def _prep_edges(edge_index, edge_weight):
    pad = _NW * _EPW - _E
    src = jnp.concatenate([edge_index[0], jnp.zeros((pad,), jnp.int32)])
    dst = jnp.concatenate([edge_index[1], jnp.zeros((pad,), jnp.int32)])
    ew = jnp.concatenate([edge_weight, jnp.zeros((pad,), jnp.float32)])
    return (src.reshape(_NW, _CH, _C), dst.reshape(_NW, _CH, _C),
            ew.reshape(_NW, _CH, _C))


def kernel(x, edge_index, edge_weight, w1a, b1a, g1, be1, w1b, b1b,
           w2a, b2a, g2, be2, w2b, b2b):
    src, dst, ew = _prep_edges(edge_index, edge_weight)
    w1aU = w1a[:, _U]
    w1bU = w1b[_U, :]
    y1U, y1bf = _tc1(x, w1aU, w1a)
    parts1 = _segsum(y1bf, src, dst, ew)
    h1U, h1bf = _tc2(parts1, y1U, b1a[_U], g1[_U], be1[_U],
                     w1bU[:, _U], w1bU, b1b[_U], b1b)
    parts2 = _segsum(h1bf, src, dst, ew)
    return _tc3(parts2, h1U, w2a[_U, :], b2a, g2, be2, w2b, b2b)


# speedup vs baseline: 16.2568x; 1.1211x over previous
import functools

import jax
import jax.numpy as jnp
import numpy as np
from jax import lax
from jax.experimental import pallas as pl
from jax.experimental.pallas import tpu as pltpu
from jax.experimental.pallas import tpu_sc as plsc

_N = 10000
_E = 320000
_DIN = 128
_H = 32
_NC = 2
_NS = 16
_NW = _NC * _NS
_C = 128
_CHT = 158
_K0 = 66
_K1 = _CHT - _K0
_KMAX = max(_K0, _K1)
_NPAD = 10240
_RPT = _NPAD // _NS
_BN = float(1.0 / np.sqrt(1.0 + 1e-5))
_U = np.concatenate([np.arange(0, 32, 2), np.arange(1, 32, 2)])

_ROWB = 1000
_GRID = _N // _ROWB


def _segsum_body(y_hbm, eidx_hbm, ew_hbm, out_hbm,
                 acc_sh, src_v, dst_v, ew_v, grow0, grow1, srow0, srow1,
                 zbuf, gsem0, gsem1, ssem0, ssem1):
    cid = lax.axis_index("c")
    sid = lax.axis_index("s")
    base = cid * _K0
    nch = jnp.where(cid == 0, _K0, _K1)

    z16 = jnp.zeros((16,), jnp.float32)

    def _zero_row(i, carry):
        zbuf[i, pl.ds(0, 16)] = z16
        zbuf[i, pl.ds(16, 16)] = z16
        return carry

    lax.fori_loop(0, _RPT, _zero_row, 0)
    pltpu.sync_copy(zbuf, acc_sh.at[pl.ds(sid * _RPT, _RPT)])

    pltpu.sync_copy(eidx_hbm.at[0, sid, pl.ds(base, _KMAX)], src_v)
    pltpu.sync_copy(eidx_hbm.at[1, sid, pl.ds(base, _KMAX)], dst_v)
    pltpu.sync_copy(ew_hbm.at[sid, pl.ds(base, _KMAX)], ew_v)

    plsc.subcore_barrier()

    gbufs = ((grow0, gsem0), (grow1, gsem1))
    sbufs = ((srow0, ssem0), (srow1, ssem1))

    def _g_start(j, b):
        rows, sem = gbufs[b]
        pltpu.async_copy(y_hbm.at[src_v.at[j]], rows, sem)

    def _g_wait(j, b):
        rows, sem = gbufs[b]
        pltpu.make_async_copy(y_hbm.at[src_v.at[j]], rows, sem).wait()

    def _s_start(j, b):
        rows, sem = sbufs[b]
        pltpu.async_copy(rows, acc_sh.at[dst_v.at[j]], sem, add=True)

    def _s_wait(j, b):
        rows, sem = sbufs[b]
        pltpu.make_async_copy(rows, acc_sh.at[dst_v.at[j]], sem).wait()

    def _mul(j, b):
        grow, _ = gbufs[b]
        srow, _ = sbufs[b]

        @plsc.parallel_loop(0, _C // 16, unroll=4)
        def _scale(i):
            e0 = i * 16
            wv = ew_v[j, pl.ds(e0, 16)]
            for k in range(16):
                w = wv[k]
                a, b2 = plsc.unpack(grow[e0 + k, :],
                                    format=plsc.PackFormat.INTERLEAVED)
                srow[e0 + k, pl.ds(0, 16)] = a * w
                srow[e0 + k, pl.ds(16, 16)] = b2 * w

    _g_start(0, 0)
    _g_start(1, 1)
    for b in range(2):
        _g_wait(b, b)
        _mul(b, b)
        _g_start(b + 2, b)
        _s_start(b, b)

    def _steady(j, carry):
        for b in range(2):
            _g_wait(j + b, b)
            _s_wait(j + b - 2, b)
            _mul(j + b, b)
            _g_start(j + b + 2, b)
            _s_start(j + b, b)
        return carry

    lax.fori_loop(1, (nch - 2) // 2, lambda g, c: _steady(g * 2, c), 0)
    for b in range(2):
        _g_wait(nch - 2 + b, b)
        _s_wait(nch - 4 + b, b)
        _mul(nch - 2 + b, b)
        _s_start(nch - 2 + b, b)
    for b in range(2):
        _s_wait(nch - 2 + b, b)

    plsc.subcore_barrier()
    pltpu.sync_copy(
        acc_sh.at[pl.ds(sid * _RPT, _RPT)],
        out_hbm.at[cid, pl.ds(sid * _RPT, _RPT)],
    )


@functools.lru_cache(maxsize=1)
def _build_segsum():
    mesh = plsc.VectorSubcoreMesh(
        core_axis_name="c", subcore_axis_name="s",
        num_cores=_NC, num_subcores=_NS,
    )
    return pl.kernel(
        _segsum_body,
        out_type=jax.ShapeDtypeStruct((_NC, _NPAD, _H), jnp.float32),
        mesh=mesh,
        scratch_types=[
            pltpu.VMEM_SHARED((_NPAD, _H), jnp.float32),
            pltpu.VMEM((_KMAX, _C), jnp.int32),
            pltpu.VMEM((_KMAX, _C), jnp.int32),
            pltpu.VMEM((_KMAX, _C), jnp.float32),
            pltpu.VMEM((_C, _H), jnp.bfloat16),
            pltpu.VMEM((_C, _H), jnp.bfloat16),
            pltpu.VMEM((_C, _H), jnp.float32),
            pltpu.VMEM((_C, _H), jnp.float32),
            pltpu.VMEM((_RPT, _H), jnp.float32),
            pltpu.SemaphoreType.DMA,
            pltpu.SemaphoreType.DMA,
            pltpu.SemaphoreType.DMA,
            pltpu.SemaphoreType.DMA,
        ],
        compiler_params=pltpu.CompilerParams(
            use_tc_tiling_on_sc=False, needs_layout_passes=False),
    )


def _segsum(y, eidx, ew):
    return _build_segsum()(y, eidx, ew)


def _tc1_body(x_ref, wU_ref, w_ref, oU_ref, obf_ref):
    x = x_ref[...]
    oU_ref[...] = jnp.dot(x, wU_ref[...], preferred_element_type=jnp.float32)
    obf_ref[...] = jnp.dot(x, w_ref[...],
                           preferred_element_type=jnp.float32
                           ).astype(jnp.bfloat16)


def _tc1(x, w1aU, w1a):
    return pl.pallas_call(
        _tc1_body,
        grid=(_GRID,),
        in_specs=[
            pl.BlockSpec((_ROWB, _DIN), lambda i: (i, 0)),
            pl.BlockSpec((_DIN, _H), lambda i: (0, 0)),
            pl.BlockSpec((_DIN, _H), lambda i: (0, 0)),
        ],
        out_specs=[
            pl.BlockSpec((_ROWB, _H), lambda i: (i, 0)),
            pl.BlockSpec((_ROWB, _H), lambda i: (i, 0)),
        ],
        out_shape=[
            jax.ShapeDtypeStruct((_N, _H), jnp.float32),
            jax.ShapeDtypeStruct((_N, _H), jnp.bfloat16),
        ],
    )(x, w1aU, w1a)


def _tc2_body(p_ref, q_ref, y_ref, b1a_ref, g1_ref, be1_ref, w1bU_ref,
              w1b_ref, b1bU_ref, b1b_ref, oU_ref, obf_ref):
    t = p_ref[0] + q_ref[0] + y_ref[...] + b1a_ref[...]
    t = t * (g1_ref[...] * _BN) + be1_ref[...]
    t = jnp.maximum(t, 0.0)
    hU = jnp.dot(t, w1bU_ref[...], preferred_element_type=jnp.float32)
    oU_ref[...] = jnp.maximum(hU + b1bU_ref[...], 0.0)
    h = jnp.dot(t, w1b_ref[...], preferred_element_type=jnp.float32)
    obf_ref[...] = jnp.maximum(h + b1b_ref[...], 0.0).astype(jnp.bfloat16)


def _tc2(parts, y1U, b1aU, g1U, be1U, w1bUU, w1bU, b1bU, b1b):
    vec = pl.BlockSpec((1, _H), lambda i: (0, 0))
    mat = pl.BlockSpec((_H, _H), lambda i: (0, 0))
    return pl.pallas_call(
        _tc2_body,
        grid=(_GRID,),
        in_specs=[
            pl.BlockSpec((1, _ROWB, _H), lambda i: (0, i, 0)),
            pl.BlockSpec((1, _ROWB, _H), lambda i: (1, i, 0)),
            pl.BlockSpec((_ROWB, _H), lambda i: (i, 0)),
            vec, vec, vec, mat, mat, vec, vec,
        ],
        out_specs=[
            pl.BlockSpec((_ROWB, _H), lambda i: (i, 0)),
            pl.BlockSpec((_ROWB, _H), lambda i: (i, 0)),
        ],
        out_shape=[
            jax.ShapeDtypeStruct((_N, _H), jnp.float32),
            jax.ShapeDtypeStruct((_N, _H), jnp.bfloat16),
        ],
    )(parts, parts, y1U, b1aU.reshape(1, _H), g1U.reshape(1, _H),
      be1U.reshape(1, _H), w1bUU, w1bU, b1bU.reshape(1, _H),
      b1b.reshape(1, _H))


def _tc3_body(p_ref, q_ref, h_ref, w2a_ref, b2a_ref, g2_ref, be2_ref,
              w2b_ref, b2b_ref, o_ref):
    t = p_ref[0] + q_ref[0] + h_ref[...]
    t = jnp.dot(t, w2a_ref[...], preferred_element_type=jnp.float32)
    t = (t + b2a_ref[...]) * (g2_ref[...] * _BN) + be2_ref[...]
    t = jnp.maximum(t, 0.0)
    t = jnp.dot(t, w2b_ref[...], preferred_element_type=jnp.float32)
    o_ref[...] = t + b2b_ref[...]


def _tc3(parts, h1U, w2aU, b2a, g2, be2, w2b, b2b):
    vec = pl.BlockSpec((1, _DIN), lambda i: (0, 0))
    return pl.pallas_call(
        _tc3_body,
        grid=(_GRID,),
        in_specs=[
            pl.BlockSpec((1, _ROWB, _H), lambda i: (0, i, 0)),
            pl.BlockSpec((1, _ROWB, _H), lambda i: (1, i, 0)),
            pl.BlockSpec((_ROWB, _H), lambda i: (i, 0)),
            pl.BlockSpec((_H, _DIN), lambda i: (0, 0)),
            vec, vec, vec,
            pl.BlockSpec((_DIN, _DIN), lambda i: (0, 0)),
            vec,
        ],
        out_specs=pl.BlockSpec((_ROWB, _DIN), lambda i: (i, 0)),
        out_shape=jax.ShapeDtypeStruct((_N, _DIN), jnp.float32),
    )(parts, parts, h1U, w2aU, b2a.reshape(1, _DIN), g2.reshape(1, _DIN),
      be2.reshape(1, _DIN), w2b, b2b.reshape(1, _DIN))


def _prep_edges(edge_index, edge_weight):
    pad = 16 * _CHT * _C - _E
    eidx = jnp.pad(edge_index, ((0, 0), (0, pad))).reshape(2, 16, _CHT, _C)
    ew = jnp.pad(edge_weight, (0, pad)).reshape(16, _CHT, _C)
    return eidx, ew


def kernel(x, edge_index, edge_weight, w1a, b1a, g1, be1, w1b, b1b,
           w2a, b2a, g2, be2, w2b, b2b):
    eidx, ew = _prep_edges(edge_index, edge_weight)
    w1aU = w1a[:, _U]
    w1bU = w1b[_U, :]
    y1U, y1bf = _tc1(x, w1aU, w1a)
    parts1 = _segsum(y1bf, eidx, ew)
    h1U, h1bf = _tc2(parts1, y1U, b1a[_U], g1[_U], be1[_U],
                     w1bU[:, _U], w1bU, b1b[_U], b1b)
    parts2 = _segsum(h1bf, eidx, ew)
    return _tc3(parts2, h1U, w2a[_U, :], b2a, g2, be2, w2b, b2b)


# speedup vs baseline: 16.8553x; 1.0368x over previous
import functools

import jax
import jax.numpy as jnp
import numpy as np
from jax import lax
from jax.experimental import pallas as pl
from jax.experimental.pallas import tpu as pltpu
from jax.experimental.pallas import tpu_sc as plsc

_N = 10000
_E = 320000
_DIN = 128
_H = 32
_NC = 2
_NS = 16
_NW = _NC * _NS
_C = 128
_CHT = 158
_K0 = 74
_K1 = _CHT - _K0
_KMAX = max(_K0, _K1)
_NPAD = 10240
_RPT = _NPAD // _NS
_BN = float(1.0 / np.sqrt(1.0 + 1e-5))
_U = np.concatenate([np.arange(0, 32, 2), np.arange(1, 32, 2)])

_ROWB = 1000
_GRID = _N // _ROWB


def _segsum_body(y_hbm, eidx_hbm, ew_hbm, out_hbm,
                 acc_sh, src_v, dst_v, ew_v, grow0, grow1, srow0, srow1,
                 zbuf, gsem0, gsem1, ssem0, ssem1):
    cid = lax.axis_index("c")
    sid = lax.axis_index("s")
    base = cid * _K0
    nch = jnp.where(cid == 0, _K0, _K1)

    z16 = jnp.zeros((16,), jnp.float32)

    def _zero_row(i, carry):
        zbuf[i, pl.ds(0, 16)] = z16
        zbuf[i, pl.ds(16, 16)] = z16
        return carry

    lax.fori_loop(0, _RPT, _zero_row, 0)
    pltpu.sync_copy(zbuf, acc_sh.at[pl.ds(sid * _RPT, _RPT)])

    pltpu.sync_copy(eidx_hbm.at[0, sid, pl.ds(base, _KMAX)], src_v)
    pltpu.sync_copy(eidx_hbm.at[1, sid, pl.ds(base, _KMAX)], dst_v)
    pltpu.sync_copy(ew_hbm.at[sid, pl.ds(base, _KMAX)], ew_v)

    plsc.subcore_barrier()

    gbufs = ((grow0, gsem0), (grow1, gsem1))
    sbufs = ((srow0, ssem0), (srow1, ssem1))

    def _g_start(j, b):
        rows, sem = gbufs[b]
        pltpu.async_copy(y_hbm.at[src_v.at[j]], rows, sem)

    def _g_wait(j, b):
        rows, sem = gbufs[b]
        pltpu.make_async_copy(y_hbm.at[src_v.at[j]], rows, sem).wait()

    def _s_start(j, b):
        rows, sem = sbufs[b]
        pltpu.async_copy(rows, acc_sh.at[dst_v.at[j]], sem, add=True)

    def _s_wait(j, b):
        rows, sem = sbufs[b]
        pltpu.make_async_copy(rows, acc_sh.at[dst_v.at[j]], sem).wait()

    def _mul(j, b):
        grow, _ = gbufs[b]
        srow, _ = sbufs[b]

        @plsc.parallel_loop(0, _C // 16, unroll=4)
        def _scale(i):
            e0 = i * 16
            wv = ew_v[j, pl.ds(e0, 16)]
            for k in range(16):
                w = wv[k]
                a, b2 = plsc.unpack(grow[e0 + k, :],
                                    format=plsc.PackFormat.INTERLEAVED)
                srow[e0 + k, pl.ds(0, 16)] = a * w
                srow[e0 + k, pl.ds(16, 16)] = b2 * w

    _g_start(0, 0)
    _g_start(1, 1)
    for b in range(2):
        _g_wait(b, b)
        _mul(b, b)
        _g_start(b + 2, b)
        _s_start(b, b)

    def _steady(j, carry):
        for b in range(2):
            _g_wait(j + b, b)
            _s_wait(j + b - 2, b)
            _mul(j + b, b)
            _g_start(j + b + 2, b)
            _s_start(j + b, b)
        return carry

    lax.fori_loop(1, (nch - 2) // 2, lambda g, c: _steady(g * 2, c), 0)
    for b in range(2):
        _g_wait(nch - 2 + b, b)
        _s_wait(nch - 4 + b, b)
        _mul(nch - 2 + b, b)
        _s_start(nch - 2 + b, b)
    for b in range(2):
        _s_wait(nch - 2 + b, b)

    plsc.subcore_barrier()
    pltpu.sync_copy(
        acc_sh.at[pl.ds(sid * _RPT, _RPT)],
        out_hbm.at[cid, pl.ds(sid * _RPT, _RPT)],
    )


@functools.lru_cache(maxsize=1)
def _build_segsum():
    mesh = plsc.VectorSubcoreMesh(
        core_axis_name="c", subcore_axis_name="s",
        num_cores=_NC, num_subcores=_NS,
    )
    return pl.kernel(
        _segsum_body,
        out_type=jax.ShapeDtypeStruct((_NC, _NPAD, _H), jnp.float32),
        mesh=mesh,
        scratch_types=[
            pltpu.VMEM_SHARED((_NPAD, _H), jnp.float32),
            pltpu.VMEM((_KMAX, _C), jnp.int32),
            pltpu.VMEM((_KMAX, _C), jnp.int32),
            pltpu.VMEM((_KMAX, _C), jnp.float32),
            pltpu.VMEM((_C, _H), jnp.bfloat16),
            pltpu.VMEM((_C, _H), jnp.bfloat16),
            pltpu.VMEM((_C, _H), jnp.float32),
            pltpu.VMEM((_C, _H), jnp.float32),
            pltpu.VMEM((_RPT, _H), jnp.float32),
            pltpu.SemaphoreType.DMA,
            pltpu.SemaphoreType.DMA,
            pltpu.SemaphoreType.DMA,
            pltpu.SemaphoreType.DMA,
        ],
        compiler_params=pltpu.CompilerParams(
            use_tc_tiling_on_sc=False, needs_layout_passes=False),
    )


def _segsum(y, eidx, ew):
    return _build_segsum()(y, eidx, ew)


def _tc1_body(x_ref, wU_ref, w_ref, oU_ref, obf_ref):
    x = x_ref[...]
    oU_ref[...] = jnp.dot(x, wU_ref[...], preferred_element_type=jnp.float32)
    obf_ref[...] = jnp.dot(x, w_ref[...],
                           preferred_element_type=jnp.float32
                           ).astype(jnp.bfloat16)


def _tc1(x, w1aU, w1a):
    return pl.pallas_call(
        _tc1_body,
        grid=(_GRID,),
        in_specs=[
            pl.BlockSpec((_ROWB, _DIN), lambda i: (i, 0)),
            pl.BlockSpec((_DIN, _H), lambda i: (0, 0)),
            pl.BlockSpec((_DIN, _H), lambda i: (0, 0)),
        ],
        out_specs=[
            pl.BlockSpec((_ROWB, _H), lambda i: (i, 0)),
            pl.BlockSpec((_ROWB, _H), lambda i: (i, 0)),
        ],
        out_shape=[
            jax.ShapeDtypeStruct((_N, _H), jnp.float32),
            jax.ShapeDtypeStruct((_N, _H), jnp.bfloat16),
        ],
    )(x, w1aU, w1a)


def _tc2_body(p_ref, q_ref, y_ref, b1a_ref, g1_ref, be1_ref, w1bU_ref,
              w1b_ref, b1bU_ref, b1b_ref, oU_ref, obf_ref):
    t = p_ref[0] + q_ref[0] + y_ref[...] + b1a_ref[...]
    t = t * (g1_ref[...] * _BN) + be1_ref[...]
    t = jnp.maximum(t, 0.0)
    hU = jnp.dot(t, w1bU_ref[...], preferred_element_type=jnp.float32)
    oU_ref[...] = jnp.maximum(hU + b1bU_ref[...], 0.0)
    h = jnp.dot(t, w1b_ref[...], preferred_element_type=jnp.float32)
    obf_ref[...] = jnp.maximum(h + b1b_ref[...], 0.0).astype(jnp.bfloat16)


def _tc2(parts, y1U, b1aU, g1U, be1U, w1bUU, w1bU, b1bU, b1b):
    vec = pl.BlockSpec((1, _H), lambda i: (0, 0))
    mat = pl.BlockSpec((_H, _H), lambda i: (0, 0))
    return pl.pallas_call(
        _tc2_body,
        grid=(_GRID,),
        in_specs=[
            pl.BlockSpec((1, _ROWB, _H), lambda i: (0, i, 0)),
            pl.BlockSpec((1, _ROWB, _H), lambda i: (1, i, 0)),
            pl.BlockSpec((_ROWB, _H), lambda i: (i, 0)),
            vec, vec, vec, mat, mat, vec, vec,
        ],
        out_specs=[
            pl.BlockSpec((_ROWB, _H), lambda i: (i, 0)),
            pl.BlockSpec((_ROWB, _H), lambda i: (i, 0)),
        ],
        out_shape=[
            jax.ShapeDtypeStruct((_N, _H), jnp.float32),
            jax.ShapeDtypeStruct((_N, _H), jnp.bfloat16),
        ],
    )(parts, parts, y1U, b1aU.reshape(1, _H), g1U.reshape(1, _H),
      be1U.reshape(1, _H), w1bUU, w1bU, b1bU.reshape(1, _H),
      b1b.reshape(1, _H))


def _tc3_body(p_ref, q_ref, h_ref, w2a_ref, b2a_ref, g2_ref, be2_ref,
              w2b_ref, b2b_ref, o_ref):
    t = p_ref[0] + q_ref[0] + h_ref[...]
    t = jnp.dot(t, w2a_ref[...], preferred_element_type=jnp.float32)
    t = (t + b2a_ref[...]) * (g2_ref[...] * _BN) + be2_ref[...]
    t = jnp.maximum(t, 0.0)
    t = jnp.dot(t, w2b_ref[...], preferred_element_type=jnp.float32)
    o_ref[...] = t + b2b_ref[...]


def _tc3(parts, h1U, w2aU, b2a, g2, be2, w2b, b2b):
    vec = pl.BlockSpec((1, _DIN), lambda i: (0, 0))
    return pl.pallas_call(
        _tc3_body,
        grid=(_GRID,),
        in_specs=[
            pl.BlockSpec((1, _ROWB, _H), lambda i: (0, i, 0)),
            pl.BlockSpec((1, _ROWB, _H), lambda i: (1, i, 0)),
            pl.BlockSpec((_ROWB, _H), lambda i: (i, 0)),
            pl.BlockSpec((_H, _DIN), lambda i: (0, 0)),
            vec, vec, vec,
            pl.BlockSpec((_DIN, _DIN), lambda i: (0, 0)),
            vec,
        ],
        out_specs=pl.BlockSpec((_ROWB, _DIN), lambda i: (i, 0)),
        out_shape=jax.ShapeDtypeStruct((_N, _DIN), jnp.float32),
    )(parts, parts, h1U, w2aU, b2a.reshape(1, _DIN), g2.reshape(1, _DIN),
      be2.reshape(1, _DIN), w2b, b2b.reshape(1, _DIN))


def _prep_edges(edge_index, edge_weight):
    pad = 16 * _CHT * _C - _E
    eidx = jnp.pad(edge_index, ((0, 0), (0, pad))).reshape(2, 16, _CHT, _C)
    ew = jnp.pad(edge_weight, (0, pad)).reshape(16, _CHT, _C)
    return eidx, ew


def kernel(x, edge_index, edge_weight, w1a, b1a, g1, be1, w1b, b1b,
           w2a, b2a, g2, be2, w2b, b2b):
    eidx, ew = _prep_edges(edge_index, edge_weight)
    w1aU = w1a[:, _U]
    w1bU = w1b[_U, :]
    y1U, y1bf = _tc1(x, w1aU, w1a)
    parts1 = _segsum(y1bf, eidx, ew)
    h1U, h1bf = _tc2(parts1, y1U, b1a[_U], g1[_U], be1[_U],
                     w1bU[:, _U], w1bU, b1b[_U], b1b)
    parts2 = _segsum(h1bf, eidx, ew)
    return _tc3(parts2, h1U, w2a[_U, :], b2a, g2, be2, w2b, b2b)


# speedup vs baseline: 17.1930x; 1.0200x over previous
import functools

import jax
import jax.numpy as jnp
import numpy as np
from jax import lax
from jax.experimental import pallas as pl
from jax.experimental.pallas import tpu as pltpu
from jax.experimental.pallas import tpu_sc as plsc

_N = 10000
_E = 320000
_DIN = 128
_H = 32
_NC = 2
_NS = 16
_NW = _NC * _NS
_C = 128
_CHT = 158
_K0 = 78
_K1 = _CHT - _K0
_KMAX = max(_K0, _K1)
_NPAD = 10240
_RPT = _NPAD // _NS
_BN = float(1.0 / np.sqrt(1.0 + 1e-5))
_U = np.concatenate([np.arange(0, 32, 2), np.arange(1, 32, 2)])

_ROWB = 1000
_GRID = _N // _ROWB


def _segsum_body(y_hbm, eidx_hbm, ew_hbm, out_hbm,
                 acc_sh, src_v, dst_v, ew_v, grow0, grow1, srow0, srow1,
                 zbuf, gsem0, gsem1, ssem0, ssem1):
    cid = lax.axis_index("c")
    sid = lax.axis_index("s")
    base = cid * _K0
    nch = jnp.where(cid == 0, _K0, _K1)

    z16 = jnp.zeros((16,), jnp.float32)

    def _zero_row(i, carry):
        zbuf[i, pl.ds(0, 16)] = z16
        zbuf[i, pl.ds(16, 16)] = z16
        return carry

    lax.fori_loop(0, _RPT, _zero_row, 0)
    pltpu.sync_copy(zbuf, acc_sh.at[pl.ds(sid * _RPT, _RPT)])

    pltpu.sync_copy(eidx_hbm.at[0, sid, pl.ds(base, _KMAX)], src_v)
    pltpu.sync_copy(eidx_hbm.at[1, sid, pl.ds(base, _KMAX)], dst_v)
    pltpu.sync_copy(ew_hbm.at[sid, pl.ds(base, _KMAX)], ew_v)

    plsc.subcore_barrier()

    gbufs = ((grow0, gsem0), (grow1, gsem1))
    sbufs = ((srow0, ssem0), (srow1, ssem1))

    def _g_start(j, b):
        rows, sem = gbufs[b]
        pltpu.async_copy(y_hbm.at[src_v.at[j]], rows, sem)

    def _g_wait(j, b):
        rows, sem = gbufs[b]
        pltpu.make_async_copy(y_hbm.at[src_v.at[j]], rows, sem).wait()

    def _s_start(j, b):
        rows, sem = sbufs[b]
        pltpu.async_copy(rows, acc_sh.at[dst_v.at[j]], sem, add=True)

    def _s_wait(j, b):
        rows, sem = sbufs[b]
        pltpu.make_async_copy(rows, acc_sh.at[dst_v.at[j]], sem).wait()

    def _mul(j, b):
        grow, _ = gbufs[b]
        srow, _ = sbufs[b]

        @plsc.parallel_loop(0, _C // 16, unroll=4)
        def _scale(i):
            e0 = i * 16
            wv = ew_v[j, pl.ds(e0, 16)]
            for k in range(16):
                w = wv[k]
                a, b2 = plsc.unpack(grow[e0 + k, :],
                                    format=plsc.PackFormat.INTERLEAVED)
                srow[e0 + k, pl.ds(0, 16)] = a * w
                srow[e0 + k, pl.ds(16, 16)] = b2 * w

    _g_start(0, 0)
    _g_start(1, 1)
    for b in range(2):
        _g_wait(b, b)
        _mul(b, b)
        _g_start(b + 2, b)
        _s_start(b, b)

    def _steady(j, carry):
        for b in range(2):
            _g_wait(j + b, b)
            _s_wait(j + b - 2, b)
            _mul(j + b, b)
            _g_start(j + b + 2, b)
            _s_start(j + b, b)
        return carry

    lax.fori_loop(1, (nch - 2) // 2, lambda g, c: _steady(g * 2, c), 0)
    for b in range(2):
        _g_wait(nch - 2 + b, b)
        _s_wait(nch - 4 + b, b)
        _mul(nch - 2 + b, b)
        _s_start(nch - 2 + b, b)
    for b in range(2):
        _s_wait(nch - 2 + b, b)

    plsc.subcore_barrier()
    pltpu.sync_copy(
        acc_sh.at[pl.ds(sid * _RPT, _RPT)],
        out_hbm.at[cid, pl.ds(sid * _RPT, _RPT)],
    )


@functools.lru_cache(maxsize=1)
def _build_segsum():
    mesh = plsc.VectorSubcoreMesh(
        core_axis_name="c", subcore_axis_name="s",
        num_cores=_NC, num_subcores=_NS,
    )
    return pl.kernel(
        _segsum_body,
        out_type=jax.ShapeDtypeStruct((_NC, _NPAD, _H), jnp.float32),
        mesh=mesh,
        scratch_types=[
            pltpu.VMEM_SHARED((_NPAD, _H), jnp.float32),
            pltpu.VMEM((_KMAX, _C), jnp.int32),
            pltpu.VMEM((_KMAX, _C), jnp.int32),
            pltpu.VMEM((_KMAX, _C), jnp.float32),
            pltpu.VMEM((_C, _H), jnp.bfloat16),
            pltpu.VMEM((_C, _H), jnp.bfloat16),
            pltpu.VMEM((_C, _H), jnp.float32),
            pltpu.VMEM((_C, _H), jnp.float32),
            pltpu.VMEM((_RPT, _H), jnp.float32),
            pltpu.SemaphoreType.DMA,
            pltpu.SemaphoreType.DMA,
            pltpu.SemaphoreType.DMA,
            pltpu.SemaphoreType.DMA,
        ],
        compiler_params=pltpu.CompilerParams(
            use_tc_tiling_on_sc=False, needs_layout_passes=False),
    )


def _segsum(y, eidx, ew):
    return _build_segsum()(y, eidx, ew)


def _tc1_body(x_ref, wU_ref, w_ref, oU_ref, obf_ref):
    x = x_ref[...]
    oU_ref[...] = jnp.dot(x, wU_ref[...], preferred_element_type=jnp.float32)
    obf_ref[...] = jnp.dot(x, w_ref[...],
                           preferred_element_type=jnp.float32
                           ).astype(jnp.bfloat16)


def _tc1(x, w1aU, w1a):
    return pl.pallas_call(
        _tc1_body,
        grid=(_GRID,),
        in_specs=[
            pl.BlockSpec((_ROWB, _DIN), lambda i: (i, 0)),
            pl.BlockSpec((_DIN, _H), lambda i: (0, 0)),
            pl.BlockSpec((_DIN, _H), lambda i: (0, 0)),
        ],
        out_specs=[
            pl.BlockSpec((_ROWB, _H), lambda i: (i, 0)),
            pl.BlockSpec((_ROWB, _H), lambda i: (i, 0)),
        ],
        out_shape=[
            jax.ShapeDtypeStruct((_N, _H), jnp.float32),
            jax.ShapeDtypeStruct((_N, _H), jnp.bfloat16),
        ],
    )(x, w1aU, w1a)


def _tc2_body(p_ref, q_ref, y_ref, b1a_ref, g1_ref, be1_ref, w1bU_ref,
              w1b_ref, b1bU_ref, b1b_ref, oU_ref, obf_ref):
    t = p_ref[0] + q_ref[0] + y_ref[...] + b1a_ref[...]
    t = t * (g1_ref[...] * _BN) + be1_ref[...]
    t = jnp.maximum(t, 0.0)
    hU = jnp.dot(t, w1bU_ref[...], preferred_element_type=jnp.float32)
    oU_ref[...] = jnp.maximum(hU + b1bU_ref[...], 0.0)
    h = jnp.dot(t, w1b_ref[...], preferred_element_type=jnp.float32)
    obf_ref[...] = jnp.maximum(h + b1b_ref[...], 0.0).astype(jnp.bfloat16)


def _tc2(parts, y1U, b1aU, g1U, be1U, w1bUU, w1bU, b1bU, b1b):
    vec = pl.BlockSpec((1, _H), lambda i: (0, 0))
    mat = pl.BlockSpec((_H, _H), lambda i: (0, 0))
    return pl.pallas_call(
        _tc2_body,
        grid=(_GRID,),
        in_specs=[
            pl.BlockSpec((1, _ROWB, _H), lambda i: (0, i, 0)),
            pl.BlockSpec((1, _ROWB, _H), lambda i: (1, i, 0)),
            pl.BlockSpec((_ROWB, _H), lambda i: (i, 0)),
            vec, vec, vec, mat, mat, vec, vec,
        ],
        out_specs=[
            pl.BlockSpec((_ROWB, _H), lambda i: (i, 0)),
            pl.BlockSpec((_ROWB, _H), lambda i: (i, 0)),
        ],
        out_shape=[
            jax.ShapeDtypeStruct((_N, _H), jnp.float32),
            jax.ShapeDtypeStruct((_N, _H), jnp.bfloat16),
        ],
    )(parts, parts, y1U, b1aU.reshape(1, _H), g1U.reshape(1, _H),
      be1U.reshape(1, _H), w1bUU, w1bU, b1bU.reshape(1, _H),
      b1b.reshape(1, _H))


def _tc3_body(p_ref, q_ref, h_ref, w2a_ref, b2a_ref, g2_ref, be2_ref,
              w2b_ref, b2b_ref, o_ref):
    t = p_ref[0] + q_ref[0] + h_ref[...]
    t = jnp.dot(t, w2a_ref[...], preferred_element_type=jnp.float32)
    t = (t + b2a_ref[...]) * (g2_ref[...] * _BN) + be2_ref[...]
    t = jnp.maximum(t, 0.0)
    t = jnp.dot(t, w2b_ref[...], preferred_element_type=jnp.float32)
    o_ref[...] = t + b2b_ref[...]


def _tc3(parts, h1U, w2aU, b2a, g2, be2, w2b, b2b):
    vec = pl.BlockSpec((1, _DIN), lambda i: (0, 0))
    return pl.pallas_call(
        _tc3_body,
        grid=(_GRID,),
        in_specs=[
            pl.BlockSpec((1, _ROWB, _H), lambda i: (0, i, 0)),
            pl.BlockSpec((1, _ROWB, _H), lambda i: (1, i, 0)),
            pl.BlockSpec((_ROWB, _H), lambda i: (i, 0)),
            pl.BlockSpec((_H, _DIN), lambda i: (0, 0)),
            vec, vec, vec,
            pl.BlockSpec((_DIN, _DIN), lambda i: (0, 0)),
            vec,
        ],
        out_specs=pl.BlockSpec((_ROWB, _DIN), lambda i: (i, 0)),
        out_shape=jax.ShapeDtypeStruct((_N, _DIN), jnp.float32),
    )(parts, parts, h1U, w2aU, b2a.reshape(1, _DIN), g2.reshape(1, _DIN),
      be2.reshape(1, _DIN), w2b, b2b.reshape(1, _DIN))


def _prep_edges(edge_index, edge_weight):
    pad = 16 * _CHT * _C - _E
    eidx = jnp.pad(edge_index, ((0, 0), (0, pad))).reshape(2, 16, _CHT, _C)
    ew = jnp.pad(edge_weight, (0, pad)).reshape(16, _CHT, _C)
    return eidx, ew


def kernel(x, edge_index, edge_weight, w1a, b1a, g1, be1, w1b, b1b,
           w2a, b2a, g2, be2, w2b, b2b):
    eidx, ew = _prep_edges(edge_index, edge_weight)
    w1aU = w1a[:, _U]
    w1bU = w1b[_U, :]
    y1U, y1bf = _tc1(x, w1aU, w1a)
    parts1 = _segsum(y1bf, eidx, ew)
    h1U, h1bf = _tc2(parts1, y1U, b1a[_U], g1[_U], be1[_U],
                     w1bU[:, _U], w1bU, b1b[_U], b1b)
    parts2 = _segsum(h1bf, eidx, ew)
    return _tc3(parts2, h1U, w2a[_U, :], b2a, g2, be2, w2b, b2b)


# speedup vs baseline: 17.5193x; 1.0190x over previous
import functools

import jax
import jax.numpy as jnp
import numpy as np
from jax import lax
from jax.experimental import pallas as pl
from jax.experimental.pallas import tpu as pltpu
from jax.experimental.pallas import tpu_sc as plsc

_N = 10000
_E = 320000
_DIN = 128
_H = 32
_NC = 2
_NS = 16
_NW = _NC * _NS
_C = 128
_CHT = 158
_K0 = 82
_K1 = _CHT - _K0
_KMAX = max(_K0, _K1)
_NPAD = 10240
_RPT = _NPAD // _NS
_BN = float(1.0 / np.sqrt(1.0 + 1e-5))
_U = np.concatenate([np.arange(0, 32, 2), np.arange(1, 32, 2)])

_ROWB = 1000
_GRID = _N // _ROWB


def _segsum_body(y_hbm, eidx_hbm, ew_hbm, out_hbm,
                 acc_sh, src_v, dst_v, ew_v, grow0, grow1, srow0, srow1,
                 zbuf, gsem0, gsem1, ssem0, ssem1):
    cid = lax.axis_index("c")
    sid = lax.axis_index("s")
    base = cid * _K0
    nch = jnp.where(cid == 0, _K0, _K1)

    z16 = jnp.zeros((16,), jnp.float32)

    def _zero_row(i, carry):
        zbuf[i, pl.ds(0, 16)] = z16
        zbuf[i, pl.ds(16, 16)] = z16
        return carry

    lax.fori_loop(0, _RPT, _zero_row, 0)
    pltpu.sync_copy(zbuf, acc_sh.at[pl.ds(sid * _RPT, _RPT)])

    pltpu.sync_copy(eidx_hbm.at[0, sid, pl.ds(base, _KMAX)], src_v)
    pltpu.sync_copy(eidx_hbm.at[1, sid, pl.ds(base, _KMAX)], dst_v)
    pltpu.sync_copy(ew_hbm.at[sid, pl.ds(base, _KMAX)], ew_v)

    plsc.subcore_barrier()

    gbufs = ((grow0, gsem0), (grow1, gsem1))
    sbufs = ((srow0, ssem0), (srow1, ssem1))

    def _g_start(j, b):
        rows, sem = gbufs[b]
        pltpu.async_copy(y_hbm.at[src_v.at[j]], rows, sem)

    def _g_wait(j, b):
        rows, sem = gbufs[b]
        pltpu.make_async_copy(y_hbm.at[src_v.at[j]], rows, sem).wait()

    def _s_start(j, b):
        rows, sem = sbufs[b]
        pltpu.async_copy(rows, acc_sh.at[dst_v.at[j]], sem, add=True)

    def _s_wait(j, b):
        rows, sem = sbufs[b]
        pltpu.make_async_copy(rows, acc_sh.at[dst_v.at[j]], sem).wait()

    def _mul(j, b):
        grow, _ = gbufs[b]
        srow, _ = sbufs[b]

        @plsc.parallel_loop(0, _C // 16, unroll=4)
        def _scale(i):
            e0 = i * 16
            wv = ew_v[j, pl.ds(e0, 16)]
            for k in range(16):
                w = wv[k]
                a, b2 = plsc.unpack(grow[e0 + k, :],
                                    format=plsc.PackFormat.INTERLEAVED)
                srow[e0 + k, pl.ds(0, 16)] = a * w
                srow[e0 + k, pl.ds(16, 16)] = b2 * w

    _g_start(0, 0)
    _g_start(1, 1)
    for b in range(2):
        _g_wait(b, b)
        _mul(b, b)
        _g_start(b + 2, b)
        _s_start(b, b)

    def _steady(j, carry):
        for b in range(2):
            _g_wait(j + b, b)
            _s_wait(j + b - 2, b)
            _mul(j + b, b)
            _g_start(j + b + 2, b)
            _s_start(j + b, b)
        return carry

    lax.fori_loop(1, (nch - 2) // 2, lambda g, c: _steady(g * 2, c), 0)
    for b in range(2):
        _g_wait(nch - 2 + b, b)
        _s_wait(nch - 4 + b, b)
        _mul(nch - 2 + b, b)
        _s_start(nch - 2 + b, b)
    for b in range(2):
        _s_wait(nch - 2 + b, b)

    plsc.subcore_barrier()
    pltpu.sync_copy(
        acc_sh.at[pl.ds(sid * _RPT, _RPT)],
        out_hbm.at[cid, pl.ds(sid * _RPT, _RPT)],
    )


@functools.lru_cache(maxsize=1)
def _build_segsum():
    mesh = plsc.VectorSubcoreMesh(
        core_axis_name="c", subcore_axis_name="s",
        num_cores=_NC, num_subcores=_NS,
    )
    return pl.kernel(
        _segsum_body,
        out_type=jax.ShapeDtypeStruct((_NC, _NPAD, _H), jnp.float32),
        mesh=mesh,
        scratch_types=[
            pltpu.VMEM_SHARED((_NPAD, _H), jnp.float32),
            pltpu.VMEM((_KMAX, _C), jnp.int32),
            pltpu.VMEM((_KMAX, _C), jnp.int32),
            pltpu.VMEM((_KMAX, _C), jnp.float32),
            pltpu.VMEM((_C, _H), jnp.bfloat16),
            pltpu.VMEM((_C, _H), jnp.bfloat16),
            pltpu.VMEM((_C, _H), jnp.float32),
            pltpu.VMEM((_C, _H), jnp.float32),
            pltpu.VMEM((_RPT, _H), jnp.float32),
            pltpu.SemaphoreType.DMA,
            pltpu.SemaphoreType.DMA,
            pltpu.SemaphoreType.DMA,
            pltpu.SemaphoreType.DMA,
        ],
        compiler_params=pltpu.CompilerParams(
            use_tc_tiling_on_sc=False, needs_layout_passes=False),
    )


def _segsum(y, eidx, ew):
    return _build_segsum()(y, eidx, ew)


def _tc1_body(x_ref, wU_ref, w_ref, oU_ref, obf_ref):
    x = x_ref[...]
    oU_ref[...] = jnp.dot(x, wU_ref[...], preferred_element_type=jnp.float32)
    obf_ref[...] = jnp.dot(x, w_ref[...],
                           preferred_element_type=jnp.float32
                           ).astype(jnp.bfloat16)


def _tc1(x, w1aU, w1a):
    return pl.pallas_call(
        _tc1_body,
        grid=(_GRID,),
        in_specs=[
            pl.BlockSpec((_ROWB, _DIN), lambda i: (i, 0)),
            pl.BlockSpec((_DIN, _H), lambda i: (0, 0)),
            pl.BlockSpec((_DIN, _H), lambda i: (0, 0)),
        ],
        out_specs=[
            pl.BlockSpec((_ROWB, _H), lambda i: (i, 0)),
            pl.BlockSpec((_ROWB, _H), lambda i: (i, 0)),
        ],
        out_shape=[
            jax.ShapeDtypeStruct((_N, _H), jnp.float32),
            jax.ShapeDtypeStruct((_N, _H), jnp.bfloat16),
        ],
    )(x, w1aU, w1a)


def _tc2_body(p_ref, q_ref, y_ref, b1a_ref, g1_ref, be1_ref, w1bU_ref,
              w1b_ref, b1bU_ref, b1b_ref, oU_ref, obf_ref):
    t = p_ref[0] + q_ref[0] + y_ref[...] + b1a_ref[...]
    t = t * (g1_ref[...] * _BN) + be1_ref[...]
    t = jnp.maximum(t, 0.0)
    hU = jnp.dot(t, w1bU_ref[...], preferred_element_type=jnp.float32)
    oU_ref[...] = jnp.maximum(hU + b1bU_ref[...], 0.0)
    h = jnp.dot(t, w1b_ref[...], preferred_element_type=jnp.float32)
    obf_ref[...] = jnp.maximum(h + b1b_ref[...], 0.0).astype(jnp.bfloat16)


def _tc2(parts, y1U, b1aU, g1U, be1U, w1bUU, w1bU, b1bU, b1b):
    vec = pl.BlockSpec((1, _H), lambda i: (0, 0))
    mat = pl.BlockSpec((_H, _H), lambda i: (0, 0))
    return pl.pallas_call(
        _tc2_body,
        grid=(_GRID,),
        in_specs=[
            pl.BlockSpec((1, _ROWB, _H), lambda i: (0, i, 0)),
            pl.BlockSpec((1, _ROWB, _H), lambda i: (1, i, 0)),
            pl.BlockSpec((_ROWB, _H), lambda i: (i, 0)),
            vec, vec, vec, mat, mat, vec, vec,
        ],
        out_specs=[
            pl.BlockSpec((_ROWB, _H), lambda i: (i, 0)),
            pl.BlockSpec((_ROWB, _H), lambda i: (i, 0)),
        ],
        out_shape=[
            jax.ShapeDtypeStruct((_N, _H), jnp.float32),
            jax.ShapeDtypeStruct((_N, _H), jnp.bfloat16),
        ],
    )(parts, parts, y1U, b1aU.reshape(1, _H), g1U.reshape(1, _H),
      be1U.reshape(1, _H), w1bUU, w1bU, b1bU.reshape(1, _H),
      b1b.reshape(1, _H))


def _tc3_body(p_ref, q_ref, h_ref, w2a_ref, b2a_ref, g2_ref, be2_ref,
              w2b_ref, b2b_ref, o_ref):
    t = p_ref[0] + q_ref[0] + h_ref[...]
    t = jnp.dot(t, w2a_ref[...], preferred_element_type=jnp.float32)
    t = (t + b2a_ref[...]) * (g2_ref[...] * _BN) + be2_ref[...]
    t = jnp.maximum(t, 0.0)
    t = jnp.dot(t, w2b_ref[...], preferred_element_type=jnp.float32)
    o_ref[...] = t + b2b_ref[...]


def _tc3(parts, h1U, w2aU, b2a, g2, be2, w2b, b2b):
    vec = pl.BlockSpec((1, _DIN), lambda i: (0, 0))
    return pl.pallas_call(
        _tc3_body,
        grid=(_GRID,),
        in_specs=[
            pl.BlockSpec((1, _ROWB, _H), lambda i: (0, i, 0)),
            pl.BlockSpec((1, _ROWB, _H), lambda i: (1, i, 0)),
            pl.BlockSpec((_ROWB, _H), lambda i: (i, 0)),
            pl.BlockSpec((_H, _DIN), lambda i: (0, 0)),
            vec, vec, vec,
            pl.BlockSpec((_DIN, _DIN), lambda i: (0, 0)),
            vec,
        ],
        out_specs=pl.BlockSpec((_ROWB, _DIN), lambda i: (i, 0)),
        out_shape=jax.ShapeDtypeStruct((_N, _DIN), jnp.float32),
    )(parts, parts, h1U, w2aU, b2a.reshape(1, _DIN), g2.reshape(1, _DIN),
      be2.reshape(1, _DIN), w2b, b2b.reshape(1, _DIN))


def _prep_edges(edge_index, edge_weight):
    pad = 16 * _CHT * _C - _E
    eidx = jnp.pad(edge_index, ((0, 0), (0, pad))).reshape(2, 16, _CHT, _C)
    ew = jnp.pad(edge_weight, (0, pad)).reshape(16, _CHT, _C)
    return eidx, ew


def kernel(x, edge_index, edge_weight, w1a, b1a, g1, be1, w1b, b1b,
           w2a, b2a, g2, be2, w2b, b2b):
    eidx, ew = _prep_edges(edge_index, edge_weight)
    w1aU = w1a[:, _U]
    w1bU = w1b[_U, :]
    y1U, y1bf = _tc1(x, w1aU, w1a)
    parts1 = _segsum(y1bf, eidx, ew)
    h1U, h1bf = _tc2(parts1, y1U, b1a[_U], g1[_U], be1[_U],
                     w1bU[:, _U], w1bU, b1b[_U], b1b)
    parts2 = _segsum(h1bf, eidx, ew)
    return _tc3(parts2, h1U, w2a[_U, :], b2a, g2, be2, w2b, b2b)


# speedup vs baseline: 17.8098x; 1.0166x over previous
import functools

import jax
import jax.numpy as jnp
import numpy as np
from jax import lax
from jax.experimental import pallas as pl
from jax.experimental.pallas import tpu as pltpu
from jax.experimental.pallas import tpu_sc as plsc

_N = 10000
_E = 320000
_DIN = 128
_H = 32
_NC = 2
_NS = 16
_NW = _NC * _NS
_C = 128
_CHT = 158
_K0 = 88
_K1 = _CHT - _K0
_KMAX = max(_K0, _K1)
_NPAD = 10240
_RPT = _NPAD // _NS
_BN = float(1.0 / np.sqrt(1.0 + 1e-5))
_U = np.concatenate([np.arange(0, 32, 2), np.arange(1, 32, 2)])

_ROWB = 1000
_GRID = _N // _ROWB


def _segsum_body(y_hbm, eidx_hbm, ew_hbm, out_hbm,
                 acc_sh, src_v, dst_v, ew_v, grow0, grow1, srow0, srow1,
                 zbuf, gsem0, gsem1, ssem0, ssem1):
    cid = lax.axis_index("c")
    sid = lax.axis_index("s")
    base = cid * _K0
    nch = jnp.where(cid == 0, _K0, _K1)

    z16 = jnp.zeros((16,), jnp.float32)

    def _zero_row(i, carry):
        zbuf[i, pl.ds(0, 16)] = z16
        zbuf[i, pl.ds(16, 16)] = z16
        return carry

    lax.fori_loop(0, _RPT, _zero_row, 0)
    pltpu.sync_copy(zbuf, acc_sh.at[pl.ds(sid * _RPT, _RPT)])

    pltpu.sync_copy(eidx_hbm.at[0, sid, pl.ds(base, _KMAX)], src_v)
    pltpu.sync_copy(eidx_hbm.at[1, sid, pl.ds(base, _KMAX)], dst_v)
    pltpu.sync_copy(ew_hbm.at[sid, pl.ds(base, _KMAX)], ew_v)

    plsc.subcore_barrier()

    gbufs = ((grow0, gsem0), (grow1, gsem1))
    sbufs = ((srow0, ssem0), (srow1, ssem1))

    def _g_start(j, b):
        rows, sem = gbufs[b]
        pltpu.async_copy(y_hbm.at[src_v.at[j]], rows, sem)

    def _g_wait(j, b):
        rows, sem = gbufs[b]
        pltpu.make_async_copy(y_hbm.at[src_v.at[j]], rows, sem).wait()

    def _s_start(j, b):
        rows, sem = sbufs[b]
        pltpu.async_copy(rows, acc_sh.at[dst_v.at[j]], sem, add=True)

    def _s_wait(j, b):
        rows, sem = sbufs[b]
        pltpu.make_async_copy(rows, acc_sh.at[dst_v.at[j]], sem).wait()

    def _mul(j, b):
        grow, _ = gbufs[b]
        srow, _ = sbufs[b]

        @plsc.parallel_loop(0, _C // 16, unroll=4)
        def _scale(i):
            e0 = i * 16
            wv = ew_v[j, pl.ds(e0, 16)]
            for k in range(16):
                w = wv[k]
                a, b2 = plsc.unpack(grow[e0 + k, :],
                                    format=plsc.PackFormat.INTERLEAVED)
                srow[e0 + k, pl.ds(0, 16)] = a * w
                srow[e0 + k, pl.ds(16, 16)] = b2 * w

    _g_start(0, 0)
    _g_start(1, 1)
    for b in range(2):
        _g_wait(b, b)
        _mul(b, b)
        _g_start(b + 2, b)
        _s_start(b, b)

    def _steady(j, carry):
        for b in range(2):
            _g_wait(j + b, b)
            _s_wait(j + b - 2, b)
            _mul(j + b, b)
            _g_start(j + b + 2, b)
            _s_start(j + b, b)
        return carry

    lax.fori_loop(1, (nch - 2) // 2, lambda g, c: _steady(g * 2, c), 0)
    for b in range(2):
        _g_wait(nch - 2 + b, b)
        _s_wait(nch - 4 + b, b)
        _mul(nch - 2 + b, b)
        _s_start(nch - 2 + b, b)
    for b in range(2):
        _s_wait(nch - 2 + b, b)

    plsc.subcore_barrier()
    pltpu.sync_copy(
        acc_sh.at[pl.ds(sid * _RPT, _RPT)],
        out_hbm.at[cid, pl.ds(sid * _RPT, _RPT)],
    )


@functools.lru_cache(maxsize=1)
def _build_segsum():
    mesh = plsc.VectorSubcoreMesh(
        core_axis_name="c", subcore_axis_name="s",
        num_cores=_NC, num_subcores=_NS,
    )
    return pl.kernel(
        _segsum_body,
        out_type=jax.ShapeDtypeStruct((_NC, _NPAD, _H), jnp.float32),
        mesh=mesh,
        scratch_types=[
            pltpu.VMEM_SHARED((_NPAD, _H), jnp.float32),
            pltpu.VMEM((_KMAX, _C), jnp.int32),
            pltpu.VMEM((_KMAX, _C), jnp.int32),
            pltpu.VMEM((_KMAX, _C), jnp.float32),
            pltpu.VMEM((_C, _H), jnp.bfloat16),
            pltpu.VMEM((_C, _H), jnp.bfloat16),
            pltpu.VMEM((_C, _H), jnp.float32),
            pltpu.VMEM((_C, _H), jnp.float32),
            pltpu.VMEM((_RPT, _H), jnp.float32),
            pltpu.SemaphoreType.DMA,
            pltpu.SemaphoreType.DMA,
            pltpu.SemaphoreType.DMA,
            pltpu.SemaphoreType.DMA,
        ],
        compiler_params=pltpu.CompilerParams(
            use_tc_tiling_on_sc=False, needs_layout_passes=False),
    )


def _segsum(y, eidx, ew):
    return _build_segsum()(y, eidx, ew)


def _tc1_body(x_ref, wU_ref, w_ref, oU_ref, obf_ref):
    x = x_ref[...]
    oU_ref[...] = jnp.dot(x, wU_ref[...], preferred_element_type=jnp.float32)
    obf_ref[...] = jnp.dot(x, w_ref[...],
                           preferred_element_type=jnp.float32
                           ).astype(jnp.bfloat16)


def _tc1(x, w1aU, w1a):
    return pl.pallas_call(
        _tc1_body,
        grid=(_GRID,),
        in_specs=[
            pl.BlockSpec((_ROWB, _DIN), lambda i: (i, 0)),
            pl.BlockSpec((_DIN, _H), lambda i: (0, 0)),
            pl.BlockSpec((_DIN, _H), lambda i: (0, 0)),
        ],
        out_specs=[
            pl.BlockSpec((_ROWB, _H), lambda i: (i, 0)),
            pl.BlockSpec((_ROWB, _H), lambda i: (i, 0)),
        ],
        out_shape=[
            jax.ShapeDtypeStruct((_N, _H), jnp.float32),
            jax.ShapeDtypeStruct((_N, _H), jnp.bfloat16),
        ],
    )(x, w1aU, w1a)


def _tc2_body(p_ref, q_ref, y_ref, b1a_ref, g1_ref, be1_ref, w1bU_ref,
              w1b_ref, b1bU_ref, b1b_ref, oU_ref, obf_ref):
    t = p_ref[0] + q_ref[0] + y_ref[...] + b1a_ref[...]
    t = t * (g1_ref[...] * _BN) + be1_ref[...]
    t = jnp.maximum(t, 0.0)
    hU = jnp.dot(t, w1bU_ref[...], preferred_element_type=jnp.float32)
    oU_ref[...] = jnp.maximum(hU + b1bU_ref[...], 0.0)
    h = jnp.dot(t, w1b_ref[...], preferred_element_type=jnp.float32)
    obf_ref[...] = jnp.maximum(h + b1b_ref[...], 0.0).astype(jnp.bfloat16)


def _tc2(parts, y1U, b1aU, g1U, be1U, w1bUU, w1bU, b1bU, b1b):
    vec = pl.BlockSpec((1, _H), lambda i: (0, 0))
    mat = pl.BlockSpec((_H, _H), lambda i: (0, 0))
    return pl.pallas_call(
        _tc2_body,
        grid=(_GRID,),
        in_specs=[
            pl.BlockSpec((1, _ROWB, _H), lambda i: (0, i, 0)),
            pl.BlockSpec((1, _ROWB, _H), lambda i: (1, i, 0)),
            pl.BlockSpec((_ROWB, _H), lambda i: (i, 0)),
            vec, vec, vec, mat, mat, vec, vec,
        ],
        out_specs=[
            pl.BlockSpec((_ROWB, _H), lambda i: (i, 0)),
            pl.BlockSpec((_ROWB, _H), lambda i: (i, 0)),
        ],
        out_shape=[
            jax.ShapeDtypeStruct((_N, _H), jnp.float32),
            jax.ShapeDtypeStruct((_N, _H), jnp.bfloat16),
        ],
    )(parts, parts, y1U, b1aU.reshape(1, _H), g1U.reshape(1, _H),
      be1U.reshape(1, _H), w1bUU, w1bU, b1bU.reshape(1, _H),
      b1b.reshape(1, _H))


def _tc3_body(p_ref, q_ref, h_ref, w2a_ref, b2a_ref, g2_ref, be2_ref,
              w2b_ref, b2b_ref, o_ref):
    t = p_ref[0] + q_ref[0] + h_ref[...]
    t = jnp.dot(t, w2a_ref[...], preferred_element_type=jnp.float32)
    t = (t + b2a_ref[...]) * (g2_ref[...] * _BN) + be2_ref[...]
    t = jnp.maximum(t, 0.0)
    t = jnp.dot(t, w2b_ref[...], preferred_element_type=jnp.float32)
    o_ref[...] = t + b2b_ref[...]


def _tc3(parts, h1U, w2aU, b2a, g2, be2, w2b, b2b):
    vec = pl.BlockSpec((1, _DIN), lambda i: (0, 0))
    return pl.pallas_call(
        _tc3_body,
        grid=(_GRID,),
        in_specs=[
            pl.BlockSpec((1, _ROWB, _H), lambda i: (0, i, 0)),
            pl.BlockSpec((1, _ROWB, _H), lambda i: (1, i, 0)),
            pl.BlockSpec((_ROWB, _H), lambda i: (i, 0)),
            pl.BlockSpec((_H, _DIN), lambda i: (0, 0)),
            vec, vec, vec,
            pl.BlockSpec((_DIN, _DIN), lambda i: (0, 0)),
            vec,
        ],
        out_specs=pl.BlockSpec((_ROWB, _DIN), lambda i: (i, 0)),
        out_shape=jax.ShapeDtypeStruct((_N, _DIN), jnp.float32),
    )(parts, parts, h1U, w2aU, b2a.reshape(1, _DIN), g2.reshape(1, _DIN),
      be2.reshape(1, _DIN), w2b, b2b.reshape(1, _DIN))


def _prep_edges(edge_index, edge_weight):
    pad = 16 * _CHT * _C - _E
    eidx = jnp.pad(edge_index, ((0, 0), (0, pad))).reshape(2, 16, _CHT, _C)
    ew = jnp.pad(edge_weight, (0, pad)).reshape(16, _CHT, _C)
    return eidx, ew


def kernel(x, edge_index, edge_weight, w1a, b1a, g1, be1, w1b, b1b,
           w2a, b2a, g2, be2, w2b, b2b):
    eidx, ew = _prep_edges(edge_index, edge_weight)
    w1aU = w1a[:, _U]
    w1bU = w1b[_U, :]
    y1U, y1bf = _tc1(x, w1aU, w1a)
    parts1 = _segsum(y1bf, eidx, ew)
    h1U, h1bf = _tc2(parts1, y1U, b1a[_U], g1[_U], be1[_U],
                     w1bU[:, _U], w1bU, b1b[_U], b1b)
    parts2 = _segsum(h1bf, eidx, ew)
    return _tc3(parts2, h1U, w2a[_U, :], b2a, g2, be2, w2b, b2b)
